# Initial kernel scaffold; baseline (speedup 1.0000x reference)
import jax
import jax.numpy as jnp
from jax.experimental import pallas as pl


def kernel(embedding, edge_index, edge_type, in_proj_w, in_proj_b, out_proj_w, out_proj_b, w1, root1, b1, w2, root2, b2):
    raise NotImplementedError("write your pallas kernel here")


# speedup vs baseline: 16.1528x; 16.1528x over previous
import functools

import jax
import jax.numpy as jnp
from jax import lax
from jax.experimental import pallas as pl
from jax.experimental.pallas import tpu as pltpu
from jax.experimental.pallas import tpu_sc as plsc

NREL = 8
NC = 2
NS = 16
NW = NC * NS
LANES = 128
K = 4


def _mesh():
    return plsc.VectorSubcoreMesh(
        core_axis_name="c", subcore_axis_name="s", num_cores=NC,
        num_subcores=NS)


def _dense1(x, wvT, bv, woT, bo, w1, root1, b1):
    N, D = x.shape
    BN = 1000
    G = N // BN

    def body(x_ref, wvT_ref, bv_ref, woT_ref, bo_ref, w1_ref, r1_ref, b1_ref,
             base_ref, y0_ref, y1_ref, y2_ref):
        xb = x_ref[...]
        v = jnp.dot(xb, wvT_ref[...], preferred_element_type=jnp.float32)
        v = v + bv_ref[...]
        o = jnp.dot(v, woT_ref[...], preferred_element_type=jnp.float32)
        o = o + bo_ref[...]
        base_ref[...] = (
            jnp.dot(o, r1_ref[...], preferred_element_type=jnp.float32)
            + b1_ref[...])
        z = [jnp.dot(o, w1_ref[r], preferred_element_type=jnp.float32)
             for r in range(NREL)]
        for p in range(2):
            y0_ref[p] = jnp.concatenate(
                [z[4 * p + q][:, 0:32] for q in range(4)], axis=1)
            y1_ref[p] = jnp.concatenate(
                [z[4 * p + q][:, 32:64] for q in range(4)], axis=1)
            y2_ref[p] = jnp.concatenate(
                [z[4 * p + q][:, 64:96] for q in range(4)], axis=1)

    full2 = lambda shape: pl.BlockSpec(shape, lambda i: (0, 0))
    yspec = pl.BlockSpec((2, BN, 128), lambda i: (0, i, 0))
    yshape = jax.ShapeDtypeStruct((2, N, 128), jnp.float32)
    out = pl.pallas_call(
        body,
        grid=(G,),
        in_specs=[
            pl.BlockSpec((BN, D), lambda i: (i, 0)),
            full2((D, D)), full2((1, D)), full2((D, D)), full2((1, D)),
            pl.BlockSpec((NREL, D, D), lambda i: (0, 0, 0)),
            full2((D, D)), full2((1, D)),
        ],
        out_specs=[
            pl.BlockSpec((BN, D), lambda i: (i, 0)),
            yspec, yspec, yspec,
        ],
        out_shape=[
            jax.ShapeDtypeStruct((N, D), jnp.float32),
            yshape, yshape, yshape,
        ],
    )(x, wvT, bv, woT, bo, w1, root1, b1)
    return out


def _dense2(base1, p00, p10, p01, p11, p02, p12, w2, root2, b2):
    N, D = base1.shape
    C = root2.shape[1]
    BN = 1000
    G = N // BN

    def body(base_ref, a0_ref, b0_ref, a1_ref, b1_ref, a2_ref, b2p_ref,
             w2_ref, r2_ref, b2_ref, y_ref, base2_ref):
        b = base_ref[...]
        h0 = jnp.maximum(b[:, 0:32] + a0_ref[...] + b0_ref[...], 0.0)
        h1 = jnp.maximum(b[:, 32:64] + a1_ref[...] + b1_ref[...], 0.0)
        h2 = jnp.maximum(b[:, 64:96] + a2_ref[...] + b2p_ref[...], 0.0)
        r2 = r2_ref[...]
        base2_ref[...] = (
            jnp.dot(h0, r2[0:32], preferred_element_type=jnp.float32)
            + jnp.dot(h1, r2[32:64], preferred_element_type=jnp.float32)
            + jnp.dot(h2, r2[64:96], preferred_element_type=jnp.float32)
            + b2_ref[...])
        z = []
        for r in range(NREL):
            w = w2_ref[r]
            z.append(
                jnp.dot(h0, w[0:32], preferred_element_type=jnp.float32)
                + jnp.dot(h1, w[32:64], preferred_element_type=jnp.float32)
                + jnp.dot(h2, w[64:96], preferred_element_type=jnp.float32))
        for p in range(2):
            y_ref[p] = jnp.concatenate(
                [z[4 * p + q] for q in range(4)], axis=1)

    chunk = pl.BlockSpec((BN, 32), lambda i: (i, 0))
    out = pl.pallas_call(
        body,
        grid=(G,),
        in_specs=[
            pl.BlockSpec((BN, D), lambda i: (i, 0)),
            chunk, chunk, chunk, chunk, chunk, chunk,
            pl.BlockSpec((NREL, D, C), lambda i: (0, 0, 0)),
            pl.BlockSpec((D, C), lambda i: (0, 0)),
            pl.BlockSpec((1, C), lambda i: (0, 0)),
        ],
        out_specs=[
            pl.BlockSpec((2, BN, 128), lambda i: (0, i, 0)),
            pl.BlockSpec((BN, C), lambda i: (i, 0)),
        ],
        out_shape=[
            jax.ShapeDtypeStruct((2, N, 128), jnp.float32),
            jax.ShapeDtypeStruct((N, C), jnp.float32),
        ],
    )(base1, p00, p10, p01, p11, p02, p12, w2, root2, b2)
    return out


def _final(base2, q0, q1):
    N, C = base2.shape
    BN = 1000
    G = N // BN

    def body(b_ref, q0_ref, q1_ref, o_ref):
        o_ref[...] = jax.nn.sigmoid(b_ref[...] + q0_ref[...] + q1_ref[...])

    spec = pl.BlockSpec((BN, C), lambda i: (i, 0))
    return pl.pallas_call(
        body,
        grid=(G,),
        in_specs=[spec, spec, spec],
        out_specs=spec,
        out_shape=jax.ShapeDtypeStruct((N, C), jnp.float32),
    )(base2, q0, q1)


def _sc_hist(dst2, rel2, ones2, nbins):
    RT = dst2.shape[0]
    RW = RT // NW
    NI = RW // K
    per_tile = nbins // NS
    SB = 2000

    @functools.partial(
        pl.kernel,
        out_type=jax.ShapeDtypeStruct((NC * nbins,), jnp.float32),
        mesh=_mesh(),
        compiler_params=pltpu.CompilerParams(use_tc_tiling_on_sc=False),
        scratch_types=[
            pltpu.VMEM((K, LANES), jnp.int32),
            pltpu.VMEM((K, LANES), jnp.int32),
            pltpu.VMEM((K, LANES), jnp.int32),
            pltpu.VMEM((K, LANES), jnp.float32),
            pltpu.VMEM((SB,), jnp.float32),
            pltpu.VMEM_SHARED((nbins,), jnp.float32),
        ],
    )
    def k(dst_h, rel_h, ones_h, out_h, dstb, relb, keyb, onesb, stage, cnt):
        c = lax.axis_index("c")
        s = lax.axis_index("s")
        w = s * NC + c

        def zstage(i, _):
            stage[pl.ds(i * 16, 16)] = jnp.zeros((16,), jnp.float32)
            return _
        lax.fori_loop(0, SB // 16, zstage, None)

        nfull = per_tile // SB
        rem = per_tile - nfull * SB

        def zbin(q, _):
            pltpu.sync_copy(stage, cnt.at[pl.ds(s * per_tile + q * SB, SB)])
            return _
        lax.fori_loop(0, nfull, zbin, None)
        if rem:
            pltpu.sync_copy(stage.at[pl.ds(0, rem)],
                            cnt.at[pl.ds(s * per_tile + nfull * SB, rem)])
        plsc.subcore_barrier()

        def it(i, _):
            r0 = w * RW + i * K
            pltpu.sync_copy(dst_h.at[pl.ds(r0, K)], dstb)
            pltpu.sync_copy(rel_h.at[pl.ds(r0, K)], relb)
            pltpu.sync_copy(ones_h.at[pl.ds(r0, K)], onesb)
            for j in range(K):
                for t in range(LANES // 16):
                    sl = pl.ds(t * 16, 16)
                    keyb[j, sl] = dstb[j, sl] * NREL + relb[j, sl]
            for j in range(K):
                pltpu.sync_copy(onesb.at[j], cnt.at[keyb.at[j]], add=True)
            return _
        lax.fori_loop(0, NI, it, None)
        plsc.subcore_barrier()

        def flush(q, _):
            off = s * per_tile + q * SB
            pltpu.sync_copy(cnt.at[pl.ds(off, SB)], stage)
            pltpu.sync_copy(stage, out_h.at[pl.ds(c * nbins + off, SB)])
            return _
        lax.fori_loop(0, nfull, flush, None)
        if rem:
            off = s * per_tile + nfull * SB
            pltpu.sync_copy(cnt.at[pl.ds(off, rem)], stage.at[pl.ds(0, rem)])
            pltpu.sync_copy(stage.at[pl.ds(0, rem)],
                            out_h.at[pl.ds(c * nbins + off, rem)])

    return k(dst2, rel2, ones2)


def _sc_norm(src2, dst2, rel2, ones2, cnt0, cnt1, n_nodes):
    RT = src2.shape[0]
    RW = RT // NW
    NI = RW // K

    @functools.partial(
        pl.kernel,
        out_type=(
            jax.ShapeDtypeStruct((RT, LANES), jnp.int32),
            jax.ShapeDtypeStruct((RT, LANES), jnp.float32),
        ),
        mesh=_mesh(),
        compiler_params=pltpu.CompilerParams(use_tc_tiling_on_sc=False),
        scratch_types=[
            pltpu.VMEM((K, LANES), jnp.int32),
            pltpu.VMEM((K, LANES), jnp.int32),
            pltpu.VMEM((K, LANES), jnp.int32),
            pltpu.VMEM((K, LANES), jnp.int32),
            pltpu.VMEM((K, LANES), jnp.float32),
            pltpu.VMEM((K, LANES), jnp.float32),
            pltpu.VMEM((K, LANES), jnp.float32),
        ],
    )
    def k(src_h, dst_h, rel_h, ones_h, cnt0_h, cnt1_h, gidx_h, norm_h,
          srcb, dstb, relb, keyb, onesb, c0b, c1b):
        c = lax.axis_index("c")
        s = lax.axis_index("s")
        w = s * NC + c

        def it(i, _):
            r0 = w * RW + i * K
            pltpu.sync_copy(src_h.at[pl.ds(r0, K)], srcb)
            pltpu.sync_copy(dst_h.at[pl.ds(r0, K)], dstb)
            pltpu.sync_copy(rel_h.at[pl.ds(r0, K)], relb)
            pltpu.sync_copy(ones_h.at[pl.ds(r0, K)], onesb)
            for j in range(K):
                for t in range(LANES // 16):
                    sl = pl.ds(t * 16, 16)
                    rv = relb[j, sl]
                    keyb[j, sl] = dstb[j, sl] * NREL + rv
                    srcb[j, sl] = ((rv >> 2) * (4 * n_nodes)
                                   + srcb[j, sl] * 4 + (rv & 3))
            for j in range(K):
                pltpu.sync_copy(cnt0_h.at[keyb.at[j]], c0b.at[j])
                pltpu.sync_copy(cnt1_h.at[keyb.at[j]], c1b.at[j])
            for j in range(K):
                for t in range(LANES // 16):
                    sl = pl.ds(t * 16, 16)
                    tot = jnp.maximum(c0b[j, sl] + c1b[j, sl], 1.0)
                    onesb[j, sl] = onesb[j, sl] / tot
            pltpu.sync_copy(srcb, gidx_h.at[pl.ds(r0, K)])
            pltpu.sync_copy(onesb, norm_h.at[pl.ds(r0, K)])
            return _
        lax.fori_loop(0, NI, it, None)

    return k(src2, dst2, rel2, ones2, cnt0, cnt1)


def _sc_scatter(y, gidx2, dst2, norm2, n_nodes):
    RT = gidx2.shape[0]
    RW = RT // NW
    NI = RW // K
    C = y.shape[1]
    rows_tile = -(-n_nodes // (NS * 8)) * 8
    n_pad = rows_tile * NS
    ZR = next(z for z in range(256, 7, -8) if rows_tile % z == 0)
    nz = rows_tile // ZR

    @functools.partial(
        pl.kernel,
        out_type=jax.ShapeDtypeStruct((NC * n_pad, C), jnp.float32),
        mesh=_mesh(),
        compiler_params=pltpu.CompilerParams(use_tc_tiling_on_sc=False),
        scratch_types=[
            pltpu.VMEM((K, LANES), jnp.int32),
            pltpu.VMEM((K, LANES), jnp.int32),
            pltpu.VMEM((K, LANES), jnp.float32),
            pltpu.VMEM((K * LANES, C), jnp.float32),
            pltpu.VMEM((ZR, C), jnp.float32),
            pltpu.VMEM_SHARED((n_pad, C), jnp.float32),
        ],
    )
    def k(y_h, gidx_h, dst_h, norm_h, out_h, gidxb, dstb, normb, msgb, zb,
          acc):
        c = lax.axis_index("c")
        s = lax.axis_index("s")
        w = s * NC + c

        def zrow(j, _):
            zb[j, pl.ds(0, 16)] = jnp.zeros((16,), jnp.float32)
            zb[j, pl.ds(16, 16)] = jnp.zeros((16,), jnp.float32)
            return _
        lax.fori_loop(0, ZR, zrow, None)

        def zacc(q, _):
            pltpu.sync_copy(zb, acc.at[pl.ds(s * rows_tile + q * ZR, ZR)])
            return _
        lax.fori_loop(0, nz, zacc, None)
        plsc.subcore_barrier()

        def it(i, _):
            r0 = w * RW + i * K
            pltpu.sync_copy(gidx_h.at[pl.ds(r0, K)], gidxb)
            pltpu.sync_copy(dst_h.at[pl.ds(r0, K)], dstb)
            pltpu.sync_copy(norm_h.at[pl.ds(r0, K)], normb)
            for j in range(K):
                pltpu.sync_copy(y_h.at[gidxb.at[j]],
                                msgb.at[pl.ds(j * LANES, LANES)])
            for j in range(K):
                def scale(t, _):
                    nv16 = normb[j, pl.ds(t * 16, 16)]
                    for l in range(16):
                        nv = nv16[l]
                        row = j * LANES + t * 16 + l
                        msgb[row, pl.ds(0, 16)] = msgb[row, pl.ds(0, 16)] * nv
                        msgb[row, pl.ds(16, 16)] = (
                            msgb[row, pl.ds(16, 16)] * nv)
                    return _
                lax.fori_loop(0, LANES // 16, scale, None)
            for j in range(K):
                pltpu.sync_copy(msgb.at[pl.ds(j * LANES, LANES)],
                                acc.at[dstb.at[j]], add=True)
            return _
        lax.fori_loop(0, NI, it, None)
        plsc.subcore_barrier()

        def flush(q, _):
            row = s * rows_tile + q * ZR
            pltpu.sync_copy(acc.at[pl.ds(row, ZR)], zb)
            pltpu.sync_copy(zb, out_h.at[pl.ds(c * n_pad + row, ZR)])
            return _
        lax.fori_loop(0, nz, flush, None)

    out = k(y, gidx2, dst2, norm2)
    return out[:n_nodes], out[n_pad:n_pad + n_nodes]


def kernel(embedding, edge_index, edge_type, in_proj_w, in_proj_b,
           out_proj_w, out_proj_b, w1, root1, b1, w2, root2, b2):
    N = embedding.shape[1]
    D = embedding.shape[2]
    E = edge_index.shape[1]
    x = embedding[0]

    wvT = in_proj_w[2 * D:3 * D].T
    bv = in_proj_b[2 * D:3 * D].reshape(1, D)
    woT = out_proj_w.T
    bo = out_proj_b.reshape(1, D)

    base1, y10, y11, y12 = _dense1(
        x, wvT, bv, woT, bo, w1, root1, b1.reshape(1, D))

    block = NW * LANES * K
    RT = -(-E // block) * (block // LANES)
    EP = RT * LANES
    pad = EP - E
    src = jnp.pad(edge_index[0], (0, pad)).reshape(RT, LANES)
    dst = jnp.pad(edge_index[1], (0, pad)).reshape(RT, LANES)
    rel = jnp.pad(edge_type, (0, pad)).reshape(RT, LANES)
    ones = jnp.pad(jnp.ones((E,), jnp.float32), (0, pad)).reshape(RT, LANES)

    nbins = N * NREL
    cnt = _sc_hist(dst, rel, ones, nbins)
    gidx, norm = _sc_norm(src, dst, rel, ones, cnt[:nbins], cnt[nbins:], N)

    p0a, p0b = _sc_scatter(y10.reshape(N * NREL, 32), gidx, dst, norm, N)
    p1a, p1b = _sc_scatter(y11.reshape(N * NREL, 32), gidx, dst, norm, N)
    p2a, p2b = _sc_scatter(y12.reshape(N * NREL, 32), gidx, dst, norm, N)

    y2, base2 = _dense2(base1, p0a, p0b, p1a, p1b, p2a, p2b,
                        w2, root2, b2.reshape(1, 32))

    qa, qb = _sc_scatter(y2.reshape(N * NREL, 32), gidx, dst, norm, N)
    return _final(base2, qa, qb)


# speedup vs baseline: 22.7372x; 1.4076x over previous
import functools

import jax
import jax.numpy as jnp
from jax import lax
from jax.experimental import pallas as pl
from jax.experimental.pallas import tpu as pltpu
from jax.experimental.pallas import tpu_sc as plsc

NREL = 8
NC = 2
NS = 16
NW = NC * NS
LANES = 128
K = 7


def _mesh():
    return plsc.VectorSubcoreMesh(
        core_axis_name="c", subcore_axis_name="s", num_cores=NC,
        num_subcores=NS)


def _dense1(x, wvT, bv, woT, bo, w1, root1, b1):
    N, D = x.shape
    BN = 1000
    G = N // BN

    def body(x_ref, wvT_ref, bv_ref, woT_ref, bo_ref, w1_ref, r1_ref, b1_ref,
             base_ref, y0_ref, y1_ref, y2_ref):
        xb = x_ref[...]
        v = jnp.dot(xb, wvT_ref[...], preferred_element_type=jnp.float32)
        v = v + bv_ref[...]
        o = jnp.dot(v, woT_ref[...], preferred_element_type=jnp.float32)
        o = o + bo_ref[...]
        base_ref[...] = (
            jnp.dot(o, r1_ref[...], preferred_element_type=jnp.float32)
            + b1_ref[...])
        z = [jnp.dot(o, w1_ref[r], preferred_element_type=jnp.float32)
             for r in range(NREL)]
        for p in range(2):
            y0_ref[p] = jnp.concatenate(
                [z[4 * p + q][:, 0:32] for q in range(4)], axis=1)
            y1_ref[p] = jnp.concatenate(
                [z[4 * p + q][:, 32:64] for q in range(4)], axis=1)
            y2_ref[p] = jnp.concatenate(
                [z[4 * p + q][:, 64:96] for q in range(4)], axis=1)

    full2 = lambda shape: pl.BlockSpec(shape, lambda i: (0, 0))
    yspec = pl.BlockSpec((2, BN, 128), lambda i: (0, i, 0))
    yshape = jax.ShapeDtypeStruct((2, N, 128), jnp.float32)
    out = pl.pallas_call(
        body,
        grid=(G,),
        in_specs=[
            pl.BlockSpec((BN, D), lambda i: (i, 0)),
            full2((D, D)), full2((1, D)), full2((D, D)), full2((1, D)),
            pl.BlockSpec((NREL, D, D), lambda i: (0, 0, 0)),
            full2((D, D)), full2((1, D)),
        ],
        out_specs=[
            pl.BlockSpec((BN, D), lambda i: (i, 0)),
            yspec, yspec, yspec,
        ],
        out_shape=[
            jax.ShapeDtypeStruct((N, D), jnp.float32),
            yshape, yshape, yshape,
        ],
    )(x, wvT, bv, woT, bo, w1, root1, b1)
    return out


def _dense2(base1, p00, p10, p01, p11, p02, p12, w2, root2, b2):
    N, D = base1.shape
    C = root2.shape[1]
    BN = 1000
    G = N // BN

    def body(base_ref, a0_ref, b0_ref, a1_ref, b1_ref, a2_ref, b2p_ref,
             w2_ref, r2_ref, b2_ref, y_ref, base2_ref):
        b = base_ref[...]
        h0 = jnp.maximum(b[:, 0:32] + a0_ref[...] + b0_ref[...], 0.0)
        h1 = jnp.maximum(b[:, 32:64] + a1_ref[...] + b1_ref[...], 0.0)
        h2 = jnp.maximum(b[:, 64:96] + a2_ref[...] + b2p_ref[...], 0.0)
        r2 = r2_ref[...]
        base2_ref[...] = (
            jnp.dot(h0, r2[0:32], preferred_element_type=jnp.float32)
            + jnp.dot(h1, r2[32:64], preferred_element_type=jnp.float32)
            + jnp.dot(h2, r2[64:96], preferred_element_type=jnp.float32)
            + b2_ref[...])
        z = []
        for r in range(NREL):
            w = w2_ref[r]
            z.append(
                jnp.dot(h0, w[0:32], preferred_element_type=jnp.float32)
                + jnp.dot(h1, w[32:64], preferred_element_type=jnp.float32)
                + jnp.dot(h2, w[64:96], preferred_element_type=jnp.float32))
        for p in range(2):
            y_ref[p] = jnp.concatenate(
                [z[4 * p + q] for q in range(4)], axis=1)

    chunk = pl.BlockSpec((BN, 32), lambda i: (i, 0))
    out = pl.pallas_call(
        body,
        grid=(G,),
        in_specs=[
            pl.BlockSpec((BN, D), lambda i: (i, 0)),
            chunk, chunk, chunk, chunk, chunk, chunk,
            pl.BlockSpec((NREL, D, C), lambda i: (0, 0, 0)),
            pl.BlockSpec((D, C), lambda i: (0, 0)),
            pl.BlockSpec((1, C), lambda i: (0, 0)),
        ],
        out_specs=[
            pl.BlockSpec((2, BN, 128), lambda i: (0, i, 0)),
            pl.BlockSpec((BN, C), lambda i: (i, 0)),
        ],
        out_shape=[
            jax.ShapeDtypeStruct((2, N, 128), jnp.float32),
            jax.ShapeDtypeStruct((N, C), jnp.float32),
        ],
    )(base1, p00, p10, p01, p11, p02, p12, w2, root2, b2)
    return out


def _final(base2, q0, q1):
    N, C = base2.shape
    BN = 1000
    G = N // BN

    def body(b_ref, q0_ref, q1_ref, o_ref):
        o_ref[...] = jax.nn.sigmoid(b_ref[...] + q0_ref[...] + q1_ref[...])

    spec = pl.BlockSpec((BN, C), lambda i: (i, 0))
    return pl.pallas_call(
        body,
        grid=(G,),
        in_specs=[spec, spec, spec],
        out_specs=spec,
        out_shape=jax.ShapeDtypeStruct((N, C), jnp.float32),
    )(base2, q0, q1)


def _sc_hist(dst2, rel2, ones2, nbins):
    RT = dst2.shape[0]
    RW = RT // NW
    NI = RW // K
    per_tile = nbins // NS
    SB = 2000

    @functools.partial(
        pl.kernel,
        out_type=jax.ShapeDtypeStruct((NC * nbins,), jnp.float32),
        mesh=_mesh(),
        compiler_params=pltpu.CompilerParams(use_tc_tiling_on_sc=False),
        scratch_types=[
            pltpu.VMEM((K, LANES), jnp.int32),
            pltpu.VMEM((K, LANES), jnp.int32),
            pltpu.VMEM((K, LANES), jnp.int32),
            pltpu.VMEM((K, LANES), jnp.float32),
            pltpu.VMEM((SB,), jnp.float32),
            pltpu.VMEM_SHARED((nbins,), jnp.float32),
        ],
    )
    def k(dst_h, rel_h, ones_h, out_h, dstb, relb, keyb, onesb, stage, cnt):
        c = lax.axis_index("c")
        s = lax.axis_index("s")
        w = s * NC + c

        def zstage(i, _):
            stage[pl.ds(i * 16, 16)] = jnp.zeros((16,), jnp.float32)
            return _
        lax.fori_loop(0, SB // 16, zstage, None)

        nfull = per_tile // SB
        rem = per_tile - nfull * SB

        def zbin(q, _):
            pltpu.sync_copy(stage, cnt.at[pl.ds(s * per_tile + q * SB, SB)])
            return _
        lax.fori_loop(0, nfull, zbin, None)
        if rem:
            pltpu.sync_copy(stage.at[pl.ds(0, rem)],
                            cnt.at[pl.ds(s * per_tile + nfull * SB, rem)])
        plsc.subcore_barrier()

        def it(i, _):
            r0 = w * RW + i * K
            pltpu.sync_copy(dst_h.at[pl.ds(r0, K)], dstb)
            pltpu.sync_copy(rel_h.at[pl.ds(r0, K)], relb)
            pltpu.sync_copy(ones_h.at[pl.ds(r0, K)], onesb)
            for j in range(K):
                for t in range(LANES // 16):
                    sl = pl.ds(t * 16, 16)
                    keyb[j, sl] = dstb[j, sl] * NREL + relb[j, sl]
            for j in range(K):
                pltpu.sync_copy(onesb.at[j], cnt.at[keyb.at[j]], add=True)
            return _
        lax.fori_loop(0, NI, it, None)
        plsc.subcore_barrier()

        def flush(q, _):
            off = s * per_tile + q * SB
            pltpu.sync_copy(cnt.at[pl.ds(off, SB)], stage)
            pltpu.sync_copy(stage, out_h.at[pl.ds(c * nbins + off, SB)])
            return _
        lax.fori_loop(0, nfull, flush, None)
        if rem:
            off = s * per_tile + nfull * SB
            pltpu.sync_copy(cnt.at[pl.ds(off, rem)], stage.at[pl.ds(0, rem)])
            pltpu.sync_copy(stage.at[pl.ds(0, rem)],
                            out_h.at[pl.ds(c * nbins + off, rem)])

    return k(dst2, rel2, ones2)


def _sc_norm(src2, dst2, rel2, ones2, cnt0, cnt1, n_nodes):
    RT = src2.shape[0]
    RW = RT // NW
    NI = RW // K

    @functools.partial(
        pl.kernel,
        out_type=(
            jax.ShapeDtypeStruct((RT, LANES), jnp.int32),
            jax.ShapeDtypeStruct((RT, LANES), jnp.float32),
        ),
        mesh=_mesh(),
        compiler_params=pltpu.CompilerParams(use_tc_tiling_on_sc=False),
        scratch_types=[
            pltpu.VMEM((K, LANES), jnp.int32),
            pltpu.VMEM((K, LANES), jnp.int32),
            pltpu.VMEM((K, LANES), jnp.int32),
            pltpu.VMEM((K, LANES), jnp.int32),
            pltpu.VMEM((K, LANES), jnp.float32),
            pltpu.VMEM((K, LANES), jnp.float32),
            pltpu.VMEM((K, LANES), jnp.float32),
        ],
    )
    def k(src_h, dst_h, rel_h, ones_h, cnt0_h, cnt1_h, gidx_h, norm_h,
          srcb, dstb, relb, keyb, onesb, c0b, c1b):
        c = lax.axis_index("c")
        s = lax.axis_index("s")
        w = s * NC + c

        def it(i, _):
            r0 = w * RW + i * K
            pltpu.sync_copy(src_h.at[pl.ds(r0, K)], srcb)
            pltpu.sync_copy(dst_h.at[pl.ds(r0, K)], dstb)
            pltpu.sync_copy(rel_h.at[pl.ds(r0, K)], relb)
            pltpu.sync_copy(ones_h.at[pl.ds(r0, K)], onesb)
            for j in range(K):
                for t in range(LANES // 16):
                    sl = pl.ds(t * 16, 16)
                    rv = relb[j, sl]
                    keyb[j, sl] = dstb[j, sl] * NREL + rv
                    srcb[j, sl] = ((rv >> 2) * (4 * n_nodes)
                                   + srcb[j, sl] * 4 + (rv & 3))
            for j in range(K):
                pltpu.sync_copy(cnt0_h.at[keyb.at[j]], c0b.at[j])
                pltpu.sync_copy(cnt1_h.at[keyb.at[j]], c1b.at[j])
            for j in range(K):
                for t in range(LANES // 16):
                    sl = pl.ds(t * 16, 16)
                    tot = jnp.maximum(c0b[j, sl] + c1b[j, sl], 1.0)
                    onesb[j, sl] = onesb[j, sl] / tot
            pltpu.sync_copy(srcb, gidx_h.at[pl.ds(r0, K)])
            pltpu.sync_copy(onesb, norm_h.at[pl.ds(r0, K)])
            return _
        lax.fori_loop(0, NI, it, None)

    return k(src2, dst2, rel2, ones2, cnt0, cnt1)


def _sc_scatter(y, gidx2, dst2, norm2, n_nodes):
    KS = 2
    RT = gidx2.shape[0]
    RW = RT // NW
    NI = RW // KS
    C = y.shape[1]
    rows_tile = -(-n_nodes // (NS * 8)) * 8
    n_pad = rows_tile * NS
    ZR = next(z for z in range(256, 7, -8) if rows_tile % z == 0)
    nz = rows_tile // ZR

    @functools.partial(
        pl.kernel,
        out_type=jax.ShapeDtypeStruct((NC * n_pad, C), jnp.float32),
        mesh=_mesh(),
        compiler_params=pltpu.CompilerParams(use_tc_tiling_on_sc=False),
        scratch_types=[
            [pltpu.VMEM((KS, LANES), jnp.int32)] * 2,
            [pltpu.VMEM((KS, LANES), jnp.int32)] * 2,
            [pltpu.VMEM((KS, LANES), jnp.float32)] * 2,
            [pltpu.VMEM((KS * LANES, C), jnp.float32)] * 2,
            pltpu.VMEM((ZR, C), jnp.float32),
            pltpu.VMEM_SHARED((n_pad, C), jnp.float32),
            [pltpu.SemaphoreType.DMA] * 2,
            [pltpu.SemaphoreType.DMA] * 2,
            [pltpu.SemaphoreType.DMA] * 2,
        ],
    )
    def k(y_h, gidx_h, dst_h, norm_h, out_h, gidxb, dstb, normb, msgb, zb,
          acc, isem, gsem, ssem):
        c = lax.axis_index("c")
        s = lax.axis_index("s")
        w = s * NC + c

        def zrow(j, _):
            zb[j, pl.ds(0, 16)] = jnp.zeros((16,), jnp.float32)
            zb[j, pl.ds(16, 16)] = jnp.zeros((16,), jnp.float32)
            return _
        lax.fori_loop(0, ZR, zrow, None)

        def zacc(q, _):
            pltpu.sync_copy(zb, acc.at[pl.ds(s * rows_tile + q * ZR, ZR)])
            return _
        lax.fori_loop(0, nz, zacc, None)
        plsc.subcore_barrier()

        def fire_idx(i, b):
            r0 = w * RW + i * KS
            pltpu.async_copy(gidx_h.at[pl.ds(r0, KS)], gidxb[b], isem[b])
            pltpu.async_copy(dst_h.at[pl.ds(r0, KS)], dstb[b], isem[b])
            pltpu.async_copy(norm_h.at[pl.ds(r0, KS)], normb[b], isem[b])

        def wait_idx(b):
            pltpu.make_async_copy(gidx_h.at[pl.ds(0, KS)], gidxb[b],
                                  isem[b]).wait()
            pltpu.make_async_copy(dst_h.at[pl.ds(0, KS)], dstb[b],
                                  isem[b]).wait()
            pltpu.make_async_copy(norm_h.at[pl.ds(0, KS)], normb[b],
                                  isem[b]).wait()

        def fire_gathers(b):
            for j in range(KS):
                pltpu.async_copy(y_h.at[gidxb[b].at[j]],
                                 msgb[b].at[pl.ds(j * LANES, LANES)], gsem[b])

        def drain_gathers(b):
            for j in range(KS):
                pltpu.make_async_copy(
                    y_h.at[gidxb[b].at[j]],
                    msgb[b].at[pl.ds(j * LANES, LANES)], gsem[b]).wait()

        def fire_scatters(b):
            for j in range(KS):
                pltpu.async_copy(msgb[b].at[pl.ds(j * LANES, LANES)],
                                 acc.at[dstb[b].at[j]], ssem[b], add=True)

        def drain_scatters(b):
            for j in range(KS):
                pltpu.make_async_copy(msgb[b].at[pl.ds(j * LANES, LANES)],
                                      acc.at[dstb[b].at[j]], ssem[b]).wait()

        def scale(b):
            for j in range(KS):
                def scale_t(t, _):
                    nv16 = normb[b][j, pl.ds(t * 16, 16)]
                    for l in range(16):
                        nv = nv16[l]
                        row = j * LANES + t * 16 + l
                        msgb[b][row, pl.ds(0, 16)] = (
                            msgb[b][row, pl.ds(0, 16)] * nv)
                        msgb[b][row, pl.ds(16, 16)] = (
                            msgb[b][row, pl.ds(16, 16)] * nv)
                    return _
                lax.fori_loop(0, LANES // 16, scale_t, None)

        fire_idx(0, 0)
        wait_idx(0)
        fire_gathers(0)

        fire_idx(1, 1)
        drain_gathers(0)
        wait_idx(1)
        fire_gathers(1)
        scale(0)
        fire_scatters(0)

        def steady(i, cur):
            nxt = 1 - cur
            drain_scatters(nxt)
            fire_idx(i + 1, nxt)
            drain_gathers(cur)
            wait_idx(nxt)
            fire_gathers(nxt)
            scale(cur)
            fire_scatters(cur)

        def pair(p, _):
            steady(1 + 2 * p, 1)
            steady(2 + 2 * p, 0)
            return _
        lax.fori_loop(0, (NI - 2) // 2, pair, None)

        drain_scatters(0)
        drain_gathers(1)
        scale(1)
        fire_scatters(1)
        drain_scatters(1)
        plsc.subcore_barrier()

        def flush(q, _):
            row = s * rows_tile + q * ZR
            pltpu.sync_copy(acc.at[pl.ds(row, ZR)], zb)
            pltpu.sync_copy(zb, out_h.at[pl.ds(c * n_pad + row, ZR)])
            return _
        lax.fori_loop(0, nz, flush, None)

    out = k(y, gidx2, dst2, norm2)
    return out[:n_nodes], out[n_pad:n_pad + n_nodes]


def kernel(embedding, edge_index, edge_type, in_proj_w, in_proj_b,
           out_proj_w, out_proj_b, w1, root1, b1, w2, root2, b2):
    N = embedding.shape[1]
    D = embedding.shape[2]
    E = edge_index.shape[1]
    x = embedding[0]

    wvT = in_proj_w[2 * D:3 * D].T
    bv = in_proj_b[2 * D:3 * D].reshape(1, D)
    woT = out_proj_w.T
    bo = out_proj_b.reshape(1, D)

    base1, y10, y11, y12 = _dense1(
        x, wvT, bv, woT, bo, w1, root1, b1.reshape(1, D))

    block = NW * LANES * K
    RT = -(-E // block) * (block // LANES)
    EP = RT * LANES
    pad = EP - E
    src = jnp.pad(edge_index[0], (0, pad)).reshape(RT, LANES)
    dst = jnp.pad(edge_index[1], (0, pad)).reshape(RT, LANES)
    rel = jnp.pad(edge_type, (0, pad)).reshape(RT, LANES)
    ones = jnp.pad(jnp.ones((E,), jnp.float32), (0, pad)).reshape(RT, LANES)

    nbins = N * NREL
    cnt = _sc_hist(dst, rel, ones, nbins)
    gidx, norm = _sc_norm(src, dst, rel, ones, cnt[:nbins], cnt[nbins:], N)

    p0a, p0b = _sc_scatter(y10.reshape(N * NREL, 32), gidx, dst, norm, N)
    p1a, p1b = _sc_scatter(y11.reshape(N * NREL, 32), gidx, dst, norm, N)
    p2a, p2b = _sc_scatter(y12.reshape(N * NREL, 32), gidx, dst, norm, N)

    y2, base2 = _dense2(base1, p0a, p0b, p1a, p1b, p2a, p2b,
                        w2, root2, b2.reshape(1, 32))

    qa, qb = _sc_scatter(y2.reshape(N * NREL, 32), gidx, dst, norm, N)
    return _final(base2, qa, qb)


# speedup vs baseline: 27.1893x; 1.1958x over previous
import functools

import jax
import jax.numpy as jnp
from jax import lax
from jax.experimental import pallas as pl
from jax.experimental.pallas import tpu as pltpu
from jax.experimental.pallas import tpu_sc as plsc

NREL = 8
NC = 2
NS = 16
NW = NC * NS
LANES = 128
K = 7


def _mesh():
    return plsc.VectorSubcoreMesh(
        core_axis_name="c", subcore_axis_name="s", num_cores=NC,
        num_subcores=NS)


def _dense1(x, wvT, bv, woT, bo, w1, root1, b1):
    N, D = x.shape
    BN = 1000
    G = N // BN

    def body(x_ref, wvT_ref, bv_ref, woT_ref, bo_ref, w1_ref, r1_ref, b1_ref,
             base_ref, y0_ref, y1_ref, y2_ref):
        xb = x_ref[...]
        v = jnp.dot(xb, wvT_ref[...], preferred_element_type=jnp.float32)
        v = v + bv_ref[...]
        o = jnp.dot(v, woT_ref[...], preferred_element_type=jnp.float32)
        o = o + bo_ref[...]
        base_ref[...] = (
            jnp.dot(o, r1_ref[...], preferred_element_type=jnp.float32)
            + b1_ref[...])
        z = [jnp.dot(o, w1_ref[r], preferred_element_type=jnp.float32)
             for r in range(NREL)]
        for p in range(2):
            y0_ref[p] = jnp.concatenate(
                [z[4 * p + q][:, 0:32] for q in range(4)], axis=1)
            y1_ref[p] = jnp.concatenate(
                [z[4 * p + q][:, 32:64] for q in range(4)], axis=1)
            y2_ref[p] = jnp.concatenate(
                [z[4 * p + q][:, 64:96] for q in range(4)], axis=1)

    full2 = lambda shape: pl.BlockSpec(shape, lambda i: (0, 0))
    yspec = pl.BlockSpec((2, BN, 128), lambda i: (0, i, 0))
    yshape = jax.ShapeDtypeStruct((2, N, 128), jnp.float32)
    out = pl.pallas_call(
        body,
        grid=(G,),
        in_specs=[
            pl.BlockSpec((BN, D), lambda i: (i, 0)),
            full2((D, D)), full2((1, D)), full2((D, D)), full2((1, D)),
            pl.BlockSpec((NREL, D, D), lambda i: (0, 0, 0)),
            full2((D, D)), full2((1, D)),
        ],
        out_specs=[
            pl.BlockSpec((BN, D), lambda i: (i, 0)),
            yspec, yspec, yspec,
        ],
        out_shape=[
            jax.ShapeDtypeStruct((N, D), jnp.float32),
            yshape, yshape, yshape,
        ],
    )(x, wvT, bv, woT, bo, w1, root1, b1)
    return out


def _dense2(base1, p00, p10, p01, p11, p02, p12, w2, root2, b2):
    N, D = base1.shape
    C = root2.shape[1]
    BN = 1000
    G = N // BN

    def body(base_ref, a0_ref, b0_ref, a1_ref, b1_ref, a2_ref, b2p_ref,
             w2_ref, r2_ref, b2_ref, y_ref, base2_ref):
        b = base_ref[...]
        h0 = jnp.maximum(b[:, 0:32] + a0_ref[...] + b0_ref[...], 0.0)
        h1 = jnp.maximum(b[:, 32:64] + a1_ref[...] + b1_ref[...], 0.0)
        h2 = jnp.maximum(b[:, 64:96] + a2_ref[...] + b2p_ref[...], 0.0)
        r2 = r2_ref[...]
        base2_ref[...] = (
            jnp.dot(h0, r2[0:32], preferred_element_type=jnp.float32)
            + jnp.dot(h1, r2[32:64], preferred_element_type=jnp.float32)
            + jnp.dot(h2, r2[64:96], preferred_element_type=jnp.float32)
            + b2_ref[...])
        z = []
        for r in range(NREL):
            w = w2_ref[r]
            z.append(
                jnp.dot(h0, w[0:32], preferred_element_type=jnp.float32)
                + jnp.dot(h1, w[32:64], preferred_element_type=jnp.float32)
                + jnp.dot(h2, w[64:96], preferred_element_type=jnp.float32))
        for p in range(2):
            y_ref[p] = jnp.concatenate(
                [z[4 * p + q] for q in range(4)], axis=1)

    chunk = pl.BlockSpec((BN, 32), lambda i: (i, 0))
    out = pl.pallas_call(
        body,
        grid=(G,),
        in_specs=[
            pl.BlockSpec((BN, D), lambda i: (i, 0)),
            chunk, chunk, chunk, chunk, chunk, chunk,
            pl.BlockSpec((NREL, D, C), lambda i: (0, 0, 0)),
            pl.BlockSpec((D, C), lambda i: (0, 0)),
            pl.BlockSpec((1, C), lambda i: (0, 0)),
        ],
        out_specs=[
            pl.BlockSpec((2, BN, 128), lambda i: (0, i, 0)),
            pl.BlockSpec((BN, C), lambda i: (i, 0)),
        ],
        out_shape=[
            jax.ShapeDtypeStruct((2, N, 128), jnp.float32),
            jax.ShapeDtypeStruct((N, C), jnp.float32),
        ],
    )(base1, p00, p10, p01, p11, p02, p12, w2, root2, b2)
    return out


def _final(base2, q0, q1):
    N, C = base2.shape
    BN = 1000
    G = N // BN

    def body(b_ref, q0_ref, q1_ref, o_ref):
        o_ref[...] = jax.nn.sigmoid(b_ref[...] + q0_ref[...] + q1_ref[...])

    spec = pl.BlockSpec((BN, C), lambda i: (i, 0))
    return pl.pallas_call(
        body,
        grid=(G,),
        in_specs=[spec, spec, spec],
        out_specs=spec,
        out_shape=jax.ShapeDtypeStruct((N, C), jnp.float32),
    )(base2, q0, q1)


def _sc_hist(dst2, rel2, ones2, nbins):
    RT = dst2.shape[0]
    RW = RT // NW
    NI = RW // K
    per_tile = nbins // NS
    SB = 2000

    @functools.partial(
        pl.kernel,
        out_type=jax.ShapeDtypeStruct((NC * nbins,), jnp.float32),
        mesh=_mesh(),
        compiler_params=pltpu.CompilerParams(use_tc_tiling_on_sc=False),
        scratch_types=[
            [pltpu.VMEM((K, LANES), jnp.int32)] * 2,
            [pltpu.VMEM((K, LANES), jnp.int32)] * 2,
            [pltpu.VMEM((K, LANES), jnp.int32)] * 2,
            [pltpu.VMEM((K, LANES), jnp.float32)] * 2,
            pltpu.VMEM((SB,), jnp.float32),
            pltpu.VMEM_SHARED((nbins,), jnp.float32),
            [pltpu.SemaphoreType.DMA] * 2,
            [pltpu.SemaphoreType.DMA] * 2,
        ],
    )
    def k(dst_h, rel_h, ones_h, out_h, dstb, relb, keyb, onesb, stage, cnt,
          isem, ssem):
        c = lax.axis_index("c")
        s = lax.axis_index("s")
        w = s * NC + c

        def zstage(i, _):
            stage[pl.ds(i * 16, 16)] = jnp.zeros((16,), jnp.float32)
            return _
        lax.fori_loop(0, SB // 16, zstage, None)

        nfull = per_tile // SB
        rem = per_tile - nfull * SB

        def zbin(q, _):
            pltpu.sync_copy(stage, cnt.at[pl.ds(s * per_tile + q * SB, SB)])
            return _
        lax.fori_loop(0, nfull, zbin, None)
        if rem:
            pltpu.sync_copy(stage.at[pl.ds(0, rem)],
                            cnt.at[pl.ds(s * per_tile + nfull * SB, rem)])
        plsc.subcore_barrier()

        def fire_idx(i, b):
            r0 = w * RW + i * K
            pltpu.async_copy(dst_h.at[pl.ds(r0, K)], dstb[b], isem[b])
            pltpu.async_copy(rel_h.at[pl.ds(r0, K)], relb[b], isem[b])
            pltpu.async_copy(ones_h.at[pl.ds(r0, K)], onesb[b], isem[b])

        def wait_idx(b):
            for _ in range(3):
                pltpu.make_async_copy(dst_h.at[pl.ds(0, K)], dstb[b],
                                      isem[b]).wait()

        def compute_keys(b):
            for j in range(K):
                for t in range(LANES // 16):
                    sl = pl.ds(t * 16, 16)
                    keyb[b][j, sl] = dstb[b][j, sl] * NREL + relb[b][j, sl]

        def fire_adds(b):
            for j in range(K):
                pltpu.async_copy(onesb[b].at[j], cnt.at[keyb[b].at[j]],
                                 ssem[b], add=True)

        def drain_adds(b):
            for j in range(K):
                pltpu.make_async_copy(onesb[b].at[j], cnt.at[keyb[b].at[j]],
                                      ssem[b]).wait()

        fire_idx(0, 0)
        fire_idx(1, 1)
        wait_idx(0)
        compute_keys(0)
        fire_adds(0)

        def steady(i, cur):
            nxt = 1 - cur
            wait_idx(cur)
            compute_keys(cur)
            drain_adds(nxt)
            fire_idx(i + 1, nxt)
            fire_adds(cur)

        def pair(p, _):
            steady(1 + 2 * p, 1)
            steady(2 + 2 * p, 0)
            return _
        lax.fori_loop(0, (NI - 2) // 2, pair, None)

        wait_idx(1)
        compute_keys(1)
        drain_adds(0)
        fire_adds(1)
        drain_adds(1)
        plsc.subcore_barrier()

        def flush(q, _):
            off = s * per_tile + q * SB
            pltpu.sync_copy(cnt.at[pl.ds(off, SB)], stage)
            pltpu.sync_copy(stage, out_h.at[pl.ds(c * nbins + off, SB)])
            return _
        lax.fori_loop(0, nfull, flush, None)
        if rem:
            off = s * per_tile + nfull * SB
            pltpu.sync_copy(cnt.at[pl.ds(off, rem)], stage.at[pl.ds(0, rem)])
            pltpu.sync_copy(stage.at[pl.ds(0, rem)],
                            out_h.at[pl.ds(c * nbins + off, rem)])

    return k(dst2, rel2, ones2)


def _sc_norm(src2, dst2, rel2, ones2, cnt0, cnt1, n_nodes):
    RT = src2.shape[0]
    RW = RT // NW
    NI = RW // K

    @functools.partial(
        pl.kernel,
        out_type=(
            jax.ShapeDtypeStruct((RT, LANES), jnp.int32),
            jax.ShapeDtypeStruct((RT, LANES), jnp.float32),
        ),
        mesh=_mesh(),
        compiler_params=pltpu.CompilerParams(use_tc_tiling_on_sc=False),
        scratch_types=[
            [pltpu.VMEM((K, LANES), jnp.int32)] * 2,
            [pltpu.VMEM((K, LANES), jnp.int32)] * 2,
            [pltpu.VMEM((K, LANES), jnp.int32)] * 2,
            [pltpu.VMEM((K, LANES), jnp.int32)] * 2,
            [pltpu.VMEM((K, LANES), jnp.float32)] * 2,
            [pltpu.VMEM((K, LANES), jnp.float32)] * 2,
            [pltpu.VMEM((K, LANES), jnp.float32)] * 2,
            [pltpu.SemaphoreType.DMA] * 2,
            [pltpu.SemaphoreType.DMA] * 2,
            [pltpu.SemaphoreType.DMA] * 2,
        ],
    )
    def k(src_h, dst_h, rel_h, ones_h, cnt0_h, cnt1_h, gidx_h, norm_h,
          srcb, dstb, relb, keyb, onesb, c0b, c1b, isem, gsem, osem):
        c = lax.axis_index("c")
        s = lax.axis_index("s")
        w = s * NC + c

        def fire_idx(i, b):
            r0 = w * RW + i * K
            pltpu.async_copy(src_h.at[pl.ds(r0, K)], srcb[b], isem[b])
            pltpu.async_copy(dst_h.at[pl.ds(r0, K)], dstb[b], isem[b])
            pltpu.async_copy(rel_h.at[pl.ds(r0, K)], relb[b], isem[b])
            pltpu.async_copy(ones_h.at[pl.ds(r0, K)], onesb[b], isem[b])

        def wait_idx(b):
            for _ in range(4):
                pltpu.make_async_copy(src_h.at[pl.ds(0, K)], srcb[b],
                                      isem[b]).wait()

        def compute_keys(b):
            for j in range(K):
                for t in range(LANES // 16):
                    sl = pl.ds(t * 16, 16)
                    rv = relb[b][j, sl]
                    keyb[b][j, sl] = dstb[b][j, sl] * NREL + rv
                    srcb[b][j, sl] = ((rv >> 2) * (4 * n_nodes)
                                      + srcb[b][j, sl] * 4 + (rv & 3))

        def fire_gathers(b):
            for j in range(K):
                pltpu.async_copy(cnt0_h.at[keyb[b].at[j]], c0b[b].at[j],
                                 gsem[b])
                pltpu.async_copy(cnt1_h.at[keyb[b].at[j]], c1b[b].at[j],
                                 gsem[b])

        def drain_gathers(b):
            for j in range(K):
                pltpu.make_async_copy(cnt0_h.at[keyb[b].at[j]],
                                      c0b[b].at[j], gsem[b]).wait()
                pltpu.make_async_copy(cnt1_h.at[keyb[b].at[j]],
                                      c1b[b].at[j], gsem[b]).wait()

        def compute_norm(b):
            for j in range(K):
                for t in range(LANES // 16):
                    sl = pl.ds(t * 16, 16)
                    tot = jnp.maximum(c0b[b][j, sl] + c1b[b][j, sl], 1.0)
                    onesb[b][j, sl] = onesb[b][j, sl] / tot

        def fire_out(i, b):
            r0 = w * RW + i * K
            pltpu.async_copy(srcb[b], gidx_h.at[pl.ds(r0, K)], osem[b])
            pltpu.async_copy(onesb[b], norm_h.at[pl.ds(r0, K)], osem[b])

        def drain_out(b):
            for _ in range(2):
                pltpu.make_async_copy(srcb[b], gidx_h.at[pl.ds(0, K)],
                                      osem[b]).wait()

        fire_idx(0, 0)
        wait_idx(0)
        compute_keys(0)
        fire_gathers(0)
        fire_idx(1, 1)
        drain_gathers(0)
        compute_norm(0)
        fire_out(0, 0)

        def steady(i, cur):
            nxt = 1 - cur
            wait_idx(cur)
            compute_keys(cur)
            fire_gathers(cur)
            drain_out(nxt)
            fire_idx(i + 1, nxt)
            drain_gathers(cur)
            compute_norm(cur)
            fire_out(i, cur)

        def pair(p, _):
            steady(1 + 2 * p, 1)
            steady(2 + 2 * p, 0)
            return _
        lax.fori_loop(0, (NI - 2) // 2, pair, None)

        wait_idx(1)
        compute_keys(1)
        fire_gathers(1)
        drain_out(0)
        drain_gathers(1)
        compute_norm(1)
        fire_out(NI - 1, 1)
        drain_out(1)

    return k(src2, dst2, rel2, ones2, cnt0, cnt1)


def _sc_scatter(y, gidx2, dst2, norm2, n_nodes):
    KS = 2
    RT = gidx2.shape[0]
    RW = RT // NW
    NI = RW // KS
    C = y.shape[1]
    rows_tile = -(-n_nodes // (NS * 8)) * 8
    n_pad = rows_tile * NS
    ZR = next(z for z in range(256, 7, -8) if rows_tile % z == 0)
    nz = rows_tile // ZR

    @functools.partial(
        pl.kernel,
        out_type=jax.ShapeDtypeStruct((NC * n_pad, C), jnp.float32),
        mesh=_mesh(),
        compiler_params=pltpu.CompilerParams(use_tc_tiling_on_sc=False),
        scratch_types=[
            [pltpu.VMEM((KS, LANES), jnp.int32)] * 2,
            [pltpu.VMEM((KS, LANES), jnp.int32)] * 2,
            [pltpu.VMEM((KS, LANES), jnp.float32)] * 2,
            [pltpu.VMEM((KS * LANES, C), jnp.float32)] * 2,
            pltpu.VMEM((ZR, C), jnp.float32),
            pltpu.VMEM_SHARED((n_pad, C), jnp.float32),
            [pltpu.SemaphoreType.DMA] * 2,
            [pltpu.SemaphoreType.DMA] * 2,
            [pltpu.SemaphoreType.DMA] * 2,
        ],
    )
    def k(y_h, gidx_h, dst_h, norm_h, out_h, gidxb, dstb, normb, msgb, zb,
          acc, isem, gsem, ssem):
        c = lax.axis_index("c")
        s = lax.axis_index("s")
        w = s * NC + c

        def zrow(j, _):
            zb[j, pl.ds(0, 16)] = jnp.zeros((16,), jnp.float32)
            zb[j, pl.ds(16, 16)] = jnp.zeros((16,), jnp.float32)
            return _
        lax.fori_loop(0, ZR, zrow, None)

        def zacc(q, _):
            pltpu.sync_copy(zb, acc.at[pl.ds(s * rows_tile + q * ZR, ZR)])
            return _
        lax.fori_loop(0, nz, zacc, None)
        plsc.subcore_barrier()

        def fire_idx(i, b):
            r0 = w * RW + i * KS
            pltpu.async_copy(gidx_h.at[pl.ds(r0, KS)], gidxb[b], isem[b])
            pltpu.async_copy(dst_h.at[pl.ds(r0, KS)], dstb[b], isem[b])
            pltpu.async_copy(norm_h.at[pl.ds(r0, KS)], normb[b], isem[b])

        def wait_idx(b):
            pltpu.make_async_copy(gidx_h.at[pl.ds(0, KS)], gidxb[b],
                                  isem[b]).wait()
            pltpu.make_async_copy(dst_h.at[pl.ds(0, KS)], dstb[b],
                                  isem[b]).wait()
            pltpu.make_async_copy(norm_h.at[pl.ds(0, KS)], normb[b],
                                  isem[b]).wait()

        def fire_gathers(b):
            for j in range(KS):
                pltpu.async_copy(y_h.at[gidxb[b].at[j]],
                                 msgb[b].at[pl.ds(j * LANES, LANES)], gsem[b])

        def drain_gathers(b):
            for j in range(KS):
                pltpu.make_async_copy(
                    y_h.at[gidxb[b].at[j]],
                    msgb[b].at[pl.ds(j * LANES, LANES)], gsem[b]).wait()

        def fire_scatters(b):
            for j in range(KS):
                pltpu.async_copy(msgb[b].at[pl.ds(j * LANES, LANES)],
                                 acc.at[dstb[b].at[j]], ssem[b], add=True)

        def drain_scatters(b):
            for j in range(KS):
                pltpu.make_async_copy(msgb[b].at[pl.ds(j * LANES, LANES)],
                                      acc.at[dstb[b].at[j]], ssem[b]).wait()

        def scale(b):
            for j in range(KS):
                def scale_t(t, _):
                    nv16 = normb[b][j, pl.ds(t * 16, 16)]
                    for l in range(16):
                        nv = nv16[l]
                        row = j * LANES + t * 16 + l
                        msgb[b][row, pl.ds(0, 16)] = (
                            msgb[b][row, pl.ds(0, 16)] * nv)
                        msgb[b][row, pl.ds(16, 16)] = (
                            msgb[b][row, pl.ds(16, 16)] * nv)
                    return _
                lax.fori_loop(0, LANES // 16, scale_t, None)

        fire_idx(0, 0)
        wait_idx(0)
        fire_gathers(0)

        fire_idx(1, 1)
        drain_gathers(0)
        wait_idx(1)
        fire_gathers(1)
        scale(0)
        fire_scatters(0)

        def steady(i, cur):
            nxt = 1 - cur
            drain_scatters(nxt)
            fire_idx(i + 1, nxt)
            drain_gathers(cur)
            wait_idx(nxt)
            fire_gathers(nxt)
            scale(cur)
            fire_scatters(cur)

        def pair(p, _):
            steady(1 + 2 * p, 1)
            steady(2 + 2 * p, 0)
            return _
        lax.fori_loop(0, (NI - 2) // 2, pair, None)

        drain_scatters(0)
        drain_gathers(1)
        scale(1)
        fire_scatters(1)
        drain_scatters(1)
        plsc.subcore_barrier()

        def flush(q, _):
            row = s * rows_tile + q * ZR
            pltpu.sync_copy(acc.at[pl.ds(row, ZR)], zb)
            pltpu.sync_copy(zb, out_h.at[pl.ds(c * n_pad + row, ZR)])
            return _
        lax.fori_loop(0, nz, flush, None)

    out = k(y, gidx2, dst2, norm2)
    return out[:n_nodes], out[n_pad:n_pad + n_nodes]


def kernel(embedding, edge_index, edge_type, in_proj_w, in_proj_b,
           out_proj_w, out_proj_b, w1, root1, b1, w2, root2, b2):
    N = embedding.shape[1]
    D = embedding.shape[2]
    E = edge_index.shape[1]
    x = embedding[0]

    wvT = in_proj_w[2 * D:3 * D].T
    bv = in_proj_b[2 * D:3 * D].reshape(1, D)
    woT = out_proj_w.T
    bo = out_proj_b.reshape(1, D)

    base1, y10, y11, y12 = _dense1(
        x, wvT, bv, woT, bo, w1, root1, b1.reshape(1, D))

    block = NW * LANES * K
    RT = -(-E // block) * (block // LANES)
    EP = RT * LANES
    pad = EP - E
    src = jnp.pad(edge_index[0], (0, pad)).reshape(RT, LANES)
    dst = jnp.pad(edge_index[1], (0, pad)).reshape(RT, LANES)
    rel = jnp.pad(edge_type, (0, pad)).reshape(RT, LANES)
    ones = jnp.pad(jnp.ones((E,), jnp.float32), (0, pad)).reshape(RT, LANES)

    nbins = N * NREL
    cnt = _sc_hist(dst, rel, ones, nbins)
    gidx, norm = _sc_norm(src, dst, rel, ones, cnt[:nbins], cnt[nbins:], N)

    p0a, p0b = _sc_scatter(y10.reshape(N * NREL, 32), gidx, dst, norm, N)
    p1a, p1b = _sc_scatter(y11.reshape(N * NREL, 32), gidx, dst, norm, N)
    p2a, p2b = _sc_scatter(y12.reshape(N * NREL, 32), gidx, dst, norm, N)

    y2, base2 = _dense2(base1, p0a, p0b, p1a, p1b, p2a, p2b,
                        w2, root2, b2.reshape(1, 32))

    qa, qb = _sc_scatter(y2.reshape(N * NREL, 32), gidx, dst, norm, N)
    return _final(base2, qa, qb)


# speedup vs baseline: 29.1538x; 1.0723x over previous
import functools

import jax
import jax.numpy as jnp
from jax import lax
from jax.experimental import pallas as pl
from jax.experimental.pallas import tpu as pltpu
from jax.experimental.pallas import tpu_sc as plsc

NREL = 8
NC = 2
NS = 16
NW = NC * NS
LANES = 128
K = 7


def _mesh():
    return plsc.VectorSubcoreMesh(
        core_axis_name="c", subcore_axis_name="s", num_cores=NC,
        num_subcores=NS)


def _dense1(x, r1eff, c1, w1eff, c1r):
    N, D = x.shape
    BN = 2000
    G = N // BN

    def body(x_ref, r1_ref, c1_ref, w1_ref, c1r_ref, base_ref,
             y0_ref, y1_ref, y2_ref):
        xb = x_ref[...]
        base_ref[...] = (
            jnp.dot(xb, r1_ref[...], preferred_element_type=jnp.float32)
            + c1_ref[...])
        z = [jnp.dot(xb, w1_ref[r], preferred_element_type=jnp.float32)
             + c1r_ref[r] for r in range(NREL)]
        for p in range(2):
            y0_ref[p] = jnp.concatenate(
                [z[4 * p + q][:, 0:32] for q in range(4)], axis=1)
            y1_ref[p] = jnp.concatenate(
                [z[4 * p + q][:, 32:64] for q in range(4)], axis=1)
            y2_ref[p] = jnp.concatenate(
                [z[4 * p + q][:, 64:96] for q in range(4)], axis=1)

    full2 = lambda shape: pl.BlockSpec(shape, lambda i: (0, 0))
    yspec = pl.BlockSpec((2, BN, 128), lambda i: (0, i, 0))
    yshape = jax.ShapeDtypeStruct((2, N, 128), jnp.float32)
    out = pl.pallas_call(
        body,
        grid=(G,),
        in_specs=[
            pl.BlockSpec((BN, D), lambda i: (i, 0)),
            full2((D, D)), full2((1, D)),
            pl.BlockSpec((NREL, D, D), lambda i: (0, 0, 0)),
            pl.BlockSpec((NREL, 1, D), lambda i: (0, 0, 0)),
        ],
        out_specs=[
            pl.BlockSpec((BN, D), lambda i: (i, 0)),
            yspec, yspec, yspec,
        ],
        out_shape=[
            jax.ShapeDtypeStruct((N, D), jnp.float32),
            yshape, yshape, yshape,
        ],
    )(x, r1eff, c1, w1eff, c1r)
    return out


def _dense2(base1, p00, p10, p01, p11, p02, p12, w2, root2, b2):
    N, D = base1.shape
    C = root2.shape[1]
    BN = 2000
    G = N // BN

    def body(base_ref, a0_ref, b0_ref, a1_ref, b1_ref, a2_ref, b2p_ref,
             w2_ref, r2_ref, b2_ref, y_ref, base2_ref):
        b = base_ref[...]
        h0 = jnp.maximum(b[:, 0:32] + a0_ref[...] + b0_ref[...], 0.0)
        h1 = jnp.maximum(b[:, 32:64] + a1_ref[...] + b1_ref[...], 0.0)
        h2 = jnp.maximum(b[:, 64:96] + a2_ref[...] + b2p_ref[...], 0.0)
        r2 = r2_ref[...]
        base2_ref[...] = (
            jnp.dot(h0, r2[0:32], preferred_element_type=jnp.float32)
            + jnp.dot(h1, r2[32:64], preferred_element_type=jnp.float32)
            + jnp.dot(h2, r2[64:96], preferred_element_type=jnp.float32)
            + b2_ref[...])
        z = []
        for r in range(NREL):
            w = w2_ref[r]
            z.append(
                jnp.dot(h0, w[0:32], preferred_element_type=jnp.float32)
                + jnp.dot(h1, w[32:64], preferred_element_type=jnp.float32)
                + jnp.dot(h2, w[64:96], preferred_element_type=jnp.float32))
        for p in range(2):
            y_ref[p] = jnp.concatenate(
                [z[4 * p + q] for q in range(4)], axis=1)

    chunk = pl.BlockSpec((BN, 32), lambda i: (i, 0))
    out = pl.pallas_call(
        body,
        grid=(G,),
        in_specs=[
            pl.BlockSpec((BN, D), lambda i: (i, 0)),
            chunk, chunk, chunk, chunk, chunk, chunk,
            pl.BlockSpec((NREL, D, C), lambda i: (0, 0, 0)),
            pl.BlockSpec((D, C), lambda i: (0, 0)),
            pl.BlockSpec((1, C), lambda i: (0, 0)),
        ],
        out_specs=[
            pl.BlockSpec((2, BN, 128), lambda i: (0, i, 0)),
            pl.BlockSpec((BN, C), lambda i: (i, 0)),
        ],
        out_shape=[
            jax.ShapeDtypeStruct((2, N, 128), jnp.float32),
            jax.ShapeDtypeStruct((N, C), jnp.float32),
        ],
    )(base1, p00, p10, p01, p11, p02, p12, w2, root2, b2)
    return out


def _final(base2, q0, q1):
    N, C = base2.shape
    BN = 2000
    G = N // BN

    def body(b_ref, q0_ref, q1_ref, o_ref):
        o_ref[...] = jax.nn.sigmoid(b_ref[...] + q0_ref[...] + q1_ref[...])

    spec = pl.BlockSpec((BN, C), lambda i: (i, 0))
    return pl.pallas_call(
        body,
        grid=(G,),
        in_specs=[spec, spec, spec],
        out_specs=spec,
        out_shape=jax.ShapeDtypeStruct((N, C), jnp.float32),
    )(base2, q0, q1)


def _sc_hist(dst2, rel2, ones2, nbins):
    RT = dst2.shape[0]
    RW = RT // NW
    NI = RW // K
    per_tile = nbins // NS
    SB = 2000

    @functools.partial(
        pl.kernel,
        out_type=jax.ShapeDtypeStruct((NC * nbins,), jnp.float32),
        mesh=_mesh(),
        compiler_params=pltpu.CompilerParams(use_tc_tiling_on_sc=False),
        scratch_types=[
            [pltpu.VMEM((K, LANES), jnp.int32)] * 2,
            [pltpu.VMEM((K, LANES), jnp.int32)] * 2,
            [pltpu.VMEM((K, LANES), jnp.int32)] * 2,
            [pltpu.VMEM((K, LANES), jnp.float32)] * 2,
            pltpu.VMEM((SB,), jnp.float32),
            pltpu.VMEM_SHARED((nbins,), jnp.float32),
            [pltpu.SemaphoreType.DMA] * 2,
            [pltpu.SemaphoreType.DMA] * 2,
        ],
    )
    def k(dst_h, rel_h, ones_h, out_h, dstb, relb, keyb, onesb, stage, cnt,
          isem, ssem):
        c = lax.axis_index("c")
        s = lax.axis_index("s")
        w = s * NC + c

        def zstage(i, _):
            stage[pl.ds(i * 16, 16)] = jnp.zeros((16,), jnp.float32)
            return _
        lax.fori_loop(0, SB // 16, zstage, None)

        nfull = per_tile // SB
        rem = per_tile - nfull * SB

        def zbin(q, _):
            pltpu.sync_copy(stage, cnt.at[pl.ds(s * per_tile + q * SB, SB)])
            return _
        lax.fori_loop(0, nfull, zbin, None)
        if rem:
            pltpu.sync_copy(stage.at[pl.ds(0, rem)],
                            cnt.at[pl.ds(s * per_tile + nfull * SB, rem)])
        plsc.subcore_barrier()

        def fire_idx(i, b):
            r0 = w * RW + i * K
            pltpu.async_copy(dst_h.at[pl.ds(r0, K)], dstb[b], isem[b])
            pltpu.async_copy(rel_h.at[pl.ds(r0, K)], relb[b], isem[b])
            pltpu.async_copy(ones_h.at[pl.ds(r0, K)], onesb[b], isem[b])

        def wait_idx(b):
            for _ in range(3):
                pltpu.make_async_copy(dst_h.at[pl.ds(0, K)], dstb[b],
                                      isem[b]).wait()

        def compute_keys(b):
            for j in range(K):
                for t in range(LANES // 16):
                    sl = pl.ds(t * 16, 16)
                    keyb[b][j, sl] = dstb[b][j, sl] * NREL + relb[b][j, sl]

        def fire_adds(b):
            for j in range(K):
                pltpu.async_copy(onesb[b].at[j], cnt.at[keyb[b].at[j]],
                                 ssem[b], add=True)

        def drain_adds(b):
            for j in range(K):
                pltpu.make_async_copy(onesb[b].at[j], cnt.at[keyb[b].at[j]],
                                      ssem[b]).wait()

        fire_idx(0, 0)
        fire_idx(1, 1)
        wait_idx(0)
        compute_keys(0)
        fire_adds(0)

        def steady(i, cur):
            nxt = 1 - cur
            wait_idx(cur)
            compute_keys(cur)
            drain_adds(nxt)
            fire_idx(i + 1, nxt)
            fire_adds(cur)

        def pair(p, _):
            steady(1 + 2 * p, 1)
            steady(2 + 2 * p, 0)
            return _
        lax.fori_loop(0, (NI - 2) // 2, pair, None)

        wait_idx(1)
        compute_keys(1)
        drain_adds(0)
        fire_adds(1)
        drain_adds(1)
        plsc.subcore_barrier()

        def flush(q, _):
            off = s * per_tile + q * SB
            pltpu.sync_copy(cnt.at[pl.ds(off, SB)], stage)
            pltpu.sync_copy(stage, out_h.at[pl.ds(c * nbins + off, SB)])
            return _
        lax.fori_loop(0, nfull, flush, None)
        if rem:
            off = s * per_tile + nfull * SB
            pltpu.sync_copy(cnt.at[pl.ds(off, rem)], stage.at[pl.ds(0, rem)])
            pltpu.sync_copy(stage.at[pl.ds(0, rem)],
                            out_h.at[pl.ds(c * nbins + off, rem)])

    return k(dst2, rel2, ones2)


def _sc_norm(src2, dst2, rel2, ones2, cnt0, cnt1, n_nodes):
    RT = src2.shape[0]
    RW = RT // NW
    NI = RW // K

    @functools.partial(
        pl.kernel,
        out_type=(
            jax.ShapeDtypeStruct((RT, LANES), jnp.int32),
            jax.ShapeDtypeStruct((RT, LANES), jnp.float32),
        ),
        mesh=_mesh(),
        compiler_params=pltpu.CompilerParams(use_tc_tiling_on_sc=False),
        scratch_types=[
            [pltpu.VMEM((K, LANES), jnp.int32)] * 2,
            [pltpu.VMEM((K, LANES), jnp.int32)] * 2,
            [pltpu.VMEM((K, LANES), jnp.int32)] * 2,
            [pltpu.VMEM((K, LANES), jnp.int32)] * 2,
            [pltpu.VMEM((K, LANES), jnp.float32)] * 2,
            [pltpu.VMEM((K, LANES), jnp.float32)] * 2,
            [pltpu.VMEM((K, LANES), jnp.float32)] * 2,
            [pltpu.SemaphoreType.DMA] * 2,
            [pltpu.SemaphoreType.DMA] * 2,
            [pltpu.SemaphoreType.DMA] * 2,
        ],
    )
    def k(src_h, dst_h, rel_h, ones_h, cnt0_h, cnt1_h, gidx_h, norm_h,
          srcb, dstb, relb, keyb, onesb, c0b, c1b, isem, gsem, osem):
        c = lax.axis_index("c")
        s = lax.axis_index("s")
        w = s * NC + c

        def fire_idx(i, b):
            r0 = w * RW + i * K
            pltpu.async_copy(src_h.at[pl.ds(r0, K)], srcb[b], isem[b])
            pltpu.async_copy(dst_h.at[pl.ds(r0, K)], dstb[b], isem[b])
            pltpu.async_copy(rel_h.at[pl.ds(r0, K)], relb[b], isem[b])
            pltpu.async_copy(ones_h.at[pl.ds(r0, K)], onesb[b], isem[b])

        def wait_idx(b):
            for _ in range(4):
                pltpu.make_async_copy(src_h.at[pl.ds(0, K)], srcb[b],
                                      isem[b]).wait()

        def compute_keys(b):
            for j in range(K):
                for t in range(LANES // 16):
                    sl = pl.ds(t * 16, 16)
                    rv = relb[b][j, sl]
                    keyb[b][j, sl] = dstb[b][j, sl] * NREL + rv
                    srcb[b][j, sl] = ((rv >> 2) * (4 * n_nodes)
                                      + srcb[b][j, sl] * 4 + (rv & 3))

        def fire_gathers(b):
            for j in range(K):
                pltpu.async_copy(cnt0_h.at[keyb[b].at[j]], c0b[b].at[j],
                                 gsem[b])
                pltpu.async_copy(cnt1_h.at[keyb[b].at[j]], c1b[b].at[j],
                                 gsem[b])

        def drain_gathers(b):
            for j in range(K):
                pltpu.make_async_copy(cnt0_h.at[keyb[b].at[j]],
                                      c0b[b].at[j], gsem[b]).wait()
                pltpu.make_async_copy(cnt1_h.at[keyb[b].at[j]],
                                      c1b[b].at[j], gsem[b]).wait()

        def compute_norm(b):
            for j in range(K):
                for t in range(LANES // 16):
                    sl = pl.ds(t * 16, 16)
                    tot = jnp.maximum(c0b[b][j, sl] + c1b[b][j, sl], 1.0)
                    onesb[b][j, sl] = onesb[b][j, sl] / tot

        def fire_out(i, b):
            r0 = w * RW + i * K
            pltpu.async_copy(srcb[b], gidx_h.at[pl.ds(r0, K)], osem[b])
            pltpu.async_copy(onesb[b], norm_h.at[pl.ds(r0, K)], osem[b])

        def drain_out(b):
            for _ in range(2):
                pltpu.make_async_copy(srcb[b], gidx_h.at[pl.ds(0, K)],
                                      osem[b]).wait()

        fire_idx(0, 0)
        wait_idx(0)
        compute_keys(0)
        fire_gathers(0)
        fire_idx(1, 1)
        drain_gathers(0)
        compute_norm(0)
        fire_out(0, 0)

        def steady(i, cur):
            nxt = 1 - cur
            wait_idx(cur)
            compute_keys(cur)
            fire_gathers(cur)
            drain_out(nxt)
            fire_idx(i + 1, nxt)
            drain_gathers(cur)
            compute_norm(cur)
            fire_out(i, cur)

        def pair(p, _):
            steady(1 + 2 * p, 1)
            steady(2 + 2 * p, 0)
            return _
        lax.fori_loop(0, (NI - 2) // 2, pair, None)

        wait_idx(1)
        compute_keys(1)
        fire_gathers(1)
        drain_out(0)
        drain_gathers(1)
        compute_norm(1)
        fire_out(NI - 1, 1)
        drain_out(1)

    return k(src2, dst2, rel2, ones2, cnt0, cnt1)


def _sc_scatter(y, gidx2, dst2, norm2, n_nodes):
    KS = 2
    RT = gidx2.shape[0]
    RW = RT // NW
    NI = RW // KS
    C = y.shape[1]
    rows_tile = -(-n_nodes // (NS * 8)) * 8
    n_pad = rows_tile * NS
    ZR = next(z for z in range(256, 7, -8) if rows_tile % z == 0)
    nz = rows_tile // ZR

    @functools.partial(
        pl.kernel,
        out_type=jax.ShapeDtypeStruct((NC * n_pad, C), jnp.float32),
        mesh=_mesh(),
        compiler_params=pltpu.CompilerParams(use_tc_tiling_on_sc=False),
        scratch_types=[
            [pltpu.VMEM((KS, LANES), jnp.int32)] * 2,
            [pltpu.VMEM((KS, LANES), jnp.int32)] * 2,
            [pltpu.VMEM((KS, LANES), jnp.float32)] * 2,
            [pltpu.VMEM((KS * LANES, C), jnp.float32)] * 2,
            pltpu.VMEM((ZR, C), jnp.float32),
            pltpu.VMEM_SHARED((n_pad, C), jnp.float32),
            [pltpu.SemaphoreType.DMA] * 2,
            [pltpu.SemaphoreType.DMA] * 2,
            [pltpu.SemaphoreType.DMA] * 2,
        ],
    )
    def k(y_h, gidx_h, dst_h, norm_h, out_h, gidxb, dstb, normb, msgb, zb,
          acc, isem, gsem, ssem):
        c = lax.axis_index("c")
        s = lax.axis_index("s")
        w = s * NC + c

        def zrow(j, _):
            zb[j, pl.ds(0, 16)] = jnp.zeros((16,), jnp.float32)
            zb[j, pl.ds(16, 16)] = jnp.zeros((16,), jnp.float32)
            return _
        lax.fori_loop(0, ZR, zrow, None)

        def zacc(q, _):
            pltpu.sync_copy(zb, acc.at[pl.ds(s * rows_tile + q * ZR, ZR)])
            return _
        lax.fori_loop(0, nz, zacc, None)
        plsc.subcore_barrier()

        def fire_idx(i, b):
            r0 = w * RW + i * KS
            pltpu.async_copy(gidx_h.at[pl.ds(r0, KS)], gidxb[b], isem[b])
            pltpu.async_copy(dst_h.at[pl.ds(r0, KS)], dstb[b], isem[b])
            pltpu.async_copy(norm_h.at[pl.ds(r0, KS)], normb[b], isem[b])

        def wait_idx(b):
            pltpu.make_async_copy(gidx_h.at[pl.ds(0, KS)], gidxb[b],
                                  isem[b]).wait()
            pltpu.make_async_copy(dst_h.at[pl.ds(0, KS)], dstb[b],
                                  isem[b]).wait()
            pltpu.make_async_copy(norm_h.at[pl.ds(0, KS)], normb[b],
                                  isem[b]).wait()

        def fire_gathers(b):
            for j in range(KS):
                pltpu.async_copy(y_h.at[gidxb[b].at[j]],
                                 msgb[b].at[pl.ds(j * LANES, LANES)], gsem[b])

        def drain_gathers(b):
            for j in range(KS):
                pltpu.make_async_copy(
                    y_h.at[gidxb[b].at[j]],
                    msgb[b].at[pl.ds(j * LANES, LANES)], gsem[b]).wait()

        def fire_scatters(b):
            for j in range(KS):
                pltpu.async_copy(msgb[b].at[pl.ds(j * LANES, LANES)],
                                 acc.at[dstb[b].at[j]], ssem[b], add=True)

        def drain_scatters(b):
            for j in range(KS):
                pltpu.make_async_copy(msgb[b].at[pl.ds(j * LANES, LANES)],
                                      acc.at[dstb[b].at[j]], ssem[b]).wait()

        def scale(b):
            for j in range(KS):
                def scale_t(t, _):
                    nv16 = normb[b][j, pl.ds(t * 16, 16)]
                    for l in range(16):
                        nv = nv16[l]
                        row = j * LANES + t * 16 + l
                        msgb[b][row, pl.ds(0, 16)] = (
                            msgb[b][row, pl.ds(0, 16)] * nv)
                        msgb[b][row, pl.ds(16, 16)] = (
                            msgb[b][row, pl.ds(16, 16)] * nv)
                    return _
                lax.fori_loop(0, LANES // 16, scale_t, None)

        fire_idx(0, 0)
        wait_idx(0)
        fire_gathers(0)

        fire_idx(1, 1)
        drain_gathers(0)
        wait_idx(1)
        fire_gathers(1)
        scale(0)
        fire_scatters(0)

        def steady(i, cur):
            nxt = 1 - cur
            drain_scatters(nxt)
            fire_idx(i + 1, nxt)
            drain_gathers(cur)
            wait_idx(nxt)
            fire_gathers(nxt)
            scale(cur)
            fire_scatters(cur)

        def pair(p, _):
            steady(1 + 2 * p, 1)
            steady(2 + 2 * p, 0)
            return _
        lax.fori_loop(0, (NI - 2) // 2, pair, None)

        drain_scatters(0)
        drain_gathers(1)
        scale(1)
        fire_scatters(1)
        drain_scatters(1)
        plsc.subcore_barrier()

        def flush(q, _):
            row = s * rows_tile + q * ZR
            pltpu.sync_copy(acc.at[pl.ds(row, ZR)], zb)
            pltpu.sync_copy(zb, out_h.at[pl.ds(c * n_pad + row, ZR)])
            return _
        lax.fori_loop(0, nz, flush, None)

    out = k(y, gidx2, dst2, norm2)
    return out[:n_nodes], out[n_pad:n_pad + n_nodes]


def kernel(embedding, edge_index, edge_type, in_proj_w, in_proj_b,
           out_proj_w, out_proj_b, w1, root1, b1, w2, root2, b2):
    N = embedding.shape[1]
    D = embedding.shape[2]
    E = edge_index.shape[1]
    x = embedding[0]

    w0 = in_proj_w[2 * D:3 * D].T @ out_proj_w.T
    b0 = (in_proj_b[2 * D:3 * D] @ out_proj_w.T + out_proj_b).reshape(1, D)
    r1eff = w0 @ root1
    c1 = b0 @ root1 + b1.reshape(1, D)
    w1eff = jnp.einsum('ab,rbc->rac', w0, w1)
    c1r = jnp.einsum('ab,rbc->rac', b0, w1)

    block = NW * LANES * K
    RT = -(-E // block) * (block // LANES)
    EP = RT * LANES
    pad = EP - E
    src = jnp.pad(edge_index[0], (0, pad)).reshape(RT, LANES)
    dst = jnp.pad(edge_index[1], (0, pad)).reshape(RT, LANES)
    rel = jnp.pad(edge_type, (0, pad)).reshape(RT, LANES)
    ones = jnp.pad(jnp.ones((E,), jnp.float32), (0, pad)).reshape(RT, LANES)

    nbins = N * NREL
    cnt = _sc_hist(dst, rel, ones, nbins)
    gidx, norm = _sc_norm(src, dst, rel, ones, cnt[:nbins], cnt[nbins:], N)

    base1, y10, y11, y12 = _dense1(x, r1eff, c1, w1eff, c1r)

    p0a, p0b = _sc_scatter(y10.reshape(N * NREL, 32), gidx, dst, norm, N)
    p1a, p1b = _sc_scatter(y11.reshape(N * NREL, 32), gidx, dst, norm, N)
    p2a, p2b = _sc_scatter(y12.reshape(N * NREL, 32), gidx, dst, norm, N)

    y2, base2 = _dense2(base1, p0a, p0b, p1a, p1b, p2a, p2b,
                        w2, root2, b2.reshape(1, 32))

    qa, qb = _sc_scatter(y2.reshape(N * NREL, 32), gidx, dst, norm, N)
    return _final(base2, qa, qb)


# speedup vs baseline: 29.5905x; 1.0150x over previous
import functools

import jax
import jax.numpy as jnp
from jax import lax
from jax.experimental import pallas as pl
from jax.experimental.pallas import tpu as pltpu
from jax.experimental.pallas import tpu_sc as plsc

NREL = 8
NC = 2
NS = 16
NW = NC * NS
LANES = 128
K = 7


def _mesh():
    return plsc.VectorSubcoreMesh(
        core_axis_name="c", subcore_axis_name="s", num_cores=NC,
        num_subcores=NS)


def _dense1(x, r1eff, c1, w1eff, c1r):
    N, D = x.shape
    BN = 2000
    G = N // BN

    def body(x_ref, r1_ref, c1_ref, w1_ref, c1r_ref, base_ref,
             y0_ref, y1_ref, y2_ref):
        xb = x_ref[...]
        base_ref[...] = (
            jnp.dot(xb, r1_ref[...], preferred_element_type=jnp.float32)
            + c1_ref[...])
        z = [jnp.dot(xb, w1_ref[r], preferred_element_type=jnp.float32)
             + c1r_ref[r] for r in range(NREL)]
        for p in range(2):
            y0_ref[p] = jnp.concatenate(
                [z[4 * p + q][:, 0:32] for q in range(4)], axis=1)
            y1_ref[p] = jnp.concatenate(
                [z[4 * p + q][:, 32:64] for q in range(4)], axis=1)
            y2_ref[p] = jnp.concatenate(
                [z[4 * p + q][:, 64:96] for q in range(4)], axis=1)

    full2 = lambda shape: pl.BlockSpec(shape, lambda i: (0, 0))
    yspec = pl.BlockSpec((2, BN, 128), lambda i: (0, i, 0))
    yshape = jax.ShapeDtypeStruct((2, N, 128), jnp.float32)
    out = pl.pallas_call(
        body,
        grid=(G,),
        in_specs=[
            pl.BlockSpec((BN, D), lambda i: (i, 0)),
            full2((D, D)), full2((1, D)),
            pl.BlockSpec((NREL, D, D), lambda i: (0, 0, 0)),
            pl.BlockSpec((NREL, 1, D), lambda i: (0, 0, 0)),
        ],
        out_specs=[
            pl.BlockSpec((BN, D), lambda i: (i, 0)),
            yspec, yspec, yspec,
        ],
        out_shape=[
            jax.ShapeDtypeStruct((N, D), jnp.float32),
            yshape, yshape, yshape,
        ],
    )(x, r1eff, c1, w1eff, c1r)
    return out


def _dense2(base1, p00, p10, p01, p11, p02, p12, w2, root2, b2):
    N, D = base1.shape
    C = root2.shape[1]
    BN = 2000
    G = N // BN

    def body(base_ref, a0_ref, b0_ref, a1_ref, b1_ref, a2_ref, b2p_ref,
             w2_ref, r2_ref, b2_ref, y_ref, base2_ref):
        b = base_ref[...]
        h0 = jnp.maximum(b[:, 0:32] + a0_ref[...] + b0_ref[...], 0.0)
        h1 = jnp.maximum(b[:, 32:64] + a1_ref[...] + b1_ref[...], 0.0)
        h2 = jnp.maximum(b[:, 64:96] + a2_ref[...] + b2p_ref[...], 0.0)
        r2 = r2_ref[...]
        base2_ref[...] = (
            jnp.dot(h0, r2[0:32], preferred_element_type=jnp.float32)
            + jnp.dot(h1, r2[32:64], preferred_element_type=jnp.float32)
            + jnp.dot(h2, r2[64:96], preferred_element_type=jnp.float32)
            + b2_ref[...])
        z = []
        for r in range(NREL):
            w = w2_ref[r]
            z.append(
                jnp.dot(h0, w[0:32], preferred_element_type=jnp.float32)
                + jnp.dot(h1, w[32:64], preferred_element_type=jnp.float32)
                + jnp.dot(h2, w[64:96], preferred_element_type=jnp.float32))
        for p in range(2):
            y_ref[p] = jnp.concatenate(
                [z[4 * p + q] for q in range(4)], axis=1)

    chunk = pl.BlockSpec((BN, 32), lambda i: (i, 0))
    out = pl.pallas_call(
        body,
        grid=(G,),
        in_specs=[
            pl.BlockSpec((BN, D), lambda i: (i, 0)),
            chunk, chunk, chunk, chunk, chunk, chunk,
            pl.BlockSpec((NREL, D, C), lambda i: (0, 0, 0)),
            pl.BlockSpec((D, C), lambda i: (0, 0)),
            pl.BlockSpec((1, C), lambda i: (0, 0)),
        ],
        out_specs=[
            pl.BlockSpec((2, BN, 128), lambda i: (0, i, 0)),
            pl.BlockSpec((BN, C), lambda i: (i, 0)),
        ],
        out_shape=[
            jax.ShapeDtypeStruct((2, N, 128), jnp.float32),
            jax.ShapeDtypeStruct((N, C), jnp.float32),
        ],
    )(base1, p00, p10, p01, p11, p02, p12, w2, root2, b2)
    return out


def _final(base2, q0, q1):
    N, C = base2.shape
    BN = 2000
    G = N // BN

    def body(b_ref, q0_ref, q1_ref, o_ref):
        o_ref[...] = jax.nn.sigmoid(b_ref[...] + q0_ref[...] + q1_ref[...])

    spec = pl.BlockSpec((BN, C), lambda i: (i, 0))
    return pl.pallas_call(
        body,
        grid=(G,),
        in_specs=[spec, spec, spec],
        out_specs=spec,
        out_shape=jax.ShapeDtypeStruct((N, C), jnp.float32),
    )(base2, q0, q1)


def _sc_hist(dst2, rel2, ones2, nbins):
    RT = dst2.shape[0]
    RW = RT // NW
    NI = RW // K
    per_tile = nbins // NS
    SB = 2000

    @functools.partial(
        pl.kernel,
        out_type=jax.ShapeDtypeStruct((NC * nbins,), jnp.float32),
        mesh=_mesh(),
        compiler_params=pltpu.CompilerParams(use_tc_tiling_on_sc=False),
        scratch_types=[
            [pltpu.VMEM((K, LANES), jnp.int32)] * 2,
            [pltpu.VMEM((K, LANES), jnp.int32)] * 2,
            [pltpu.VMEM((K, LANES), jnp.int32)] * 2,
            [pltpu.VMEM((K, LANES), jnp.float32)] * 2,
            pltpu.VMEM((SB,), jnp.float32),
            pltpu.VMEM_SHARED((nbins,), jnp.float32),
            [pltpu.SemaphoreType.DMA] * 2,
            [pltpu.SemaphoreType.DMA] * 2,
        ],
    )
    def k(dst_h, rel_h, ones_h, out_h, dstb, relb, keyb, onesb, stage, cnt,
          isem, ssem):
        c = lax.axis_index("c")
        s = lax.axis_index("s")
        w = s * NC + c

        def zstage(i, _):
            stage[pl.ds(i * 16, 16)] = jnp.zeros((16,), jnp.float32)
            return _
        lax.fori_loop(0, SB // 16, zstage, None)

        nfull = per_tile // SB
        rem = per_tile - nfull * SB

        def zbin(q, _):
            pltpu.sync_copy(stage, cnt.at[pl.ds(s * per_tile + q * SB, SB)])
            return _
        lax.fori_loop(0, nfull, zbin, None)
        if rem:
            pltpu.sync_copy(stage.at[pl.ds(0, rem)],
                            cnt.at[pl.ds(s * per_tile + nfull * SB, rem)])
        plsc.subcore_barrier()

        def fire_idx(i, b):
            r0 = w * RW + i * K
            pltpu.async_copy(dst_h.at[pl.ds(r0, K)], dstb[b], isem[b])
            pltpu.async_copy(rel_h.at[pl.ds(r0, K)], relb[b], isem[b])
            pltpu.async_copy(ones_h.at[pl.ds(r0, K)], onesb[b], isem[b])

        def wait_idx(b):
            for _ in range(3):
                pltpu.make_async_copy(dst_h.at[pl.ds(0, K)], dstb[b],
                                      isem[b]).wait()

        def compute_keys(b):
            for j in range(K):
                for t in range(LANES // 16):
                    sl = pl.ds(t * 16, 16)
                    keyb[b][j, sl] = dstb[b][j, sl] * NREL + relb[b][j, sl]

        def fire_adds(b):
            for j in range(K):
                pltpu.async_copy(onesb[b].at[j], cnt.at[keyb[b].at[j]],
                                 ssem[b], add=True)

        def drain_adds(b):
            for j in range(K):
                pltpu.make_async_copy(onesb[b].at[j], cnt.at[keyb[b].at[j]],
                                      ssem[b]).wait()

        fire_idx(0, 0)
        fire_idx(1, 1)
        wait_idx(0)
        compute_keys(0)
        fire_adds(0)

        def steady(i, cur):
            nxt = 1 - cur
            wait_idx(cur)
            compute_keys(cur)
            drain_adds(nxt)
            fire_idx(i + 1, nxt)
            fire_adds(cur)

        def pair(p, _):
            steady(1 + 2 * p, 1)
            steady(2 + 2 * p, 0)
            return _
        lax.fori_loop(0, (NI - 2) // 2, pair, None)

        wait_idx(1)
        compute_keys(1)
        drain_adds(0)
        fire_adds(1)
        drain_adds(1)
        plsc.subcore_barrier()

        def flush(q, _):
            off = s * per_tile + q * SB
            pltpu.sync_copy(cnt.at[pl.ds(off, SB)], stage)
            pltpu.sync_copy(stage, out_h.at[pl.ds(c * nbins + off, SB)])
            return _
        lax.fori_loop(0, nfull, flush, None)
        if rem:
            off = s * per_tile + nfull * SB
            pltpu.sync_copy(cnt.at[pl.ds(off, rem)], stage.at[pl.ds(0, rem)])
            pltpu.sync_copy(stage.at[pl.ds(0, rem)],
                            out_h.at[pl.ds(c * nbins + off, rem)])

    return k(dst2, rel2, ones2)


def _sc_norm(src2, dst2, rel2, ones2, cnt0, cnt1, n_nodes):
    RT = src2.shape[0]
    RW = RT // NW
    NI = RW // K

    @functools.partial(
        pl.kernel,
        out_type=(
            jax.ShapeDtypeStruct((RT, LANES), jnp.int32),
            jax.ShapeDtypeStruct((RT, LANES), jnp.float32),
        ),
        mesh=_mesh(),
        compiler_params=pltpu.CompilerParams(use_tc_tiling_on_sc=False),
        scratch_types=[
            [pltpu.VMEM((K, LANES), jnp.int32)] * 2,
            [pltpu.VMEM((K, LANES), jnp.int32)] * 2,
            [pltpu.VMEM((K, LANES), jnp.int32)] * 2,
            [pltpu.VMEM((K, LANES), jnp.int32)] * 2,
            [pltpu.VMEM((K, LANES), jnp.float32)] * 2,
            [pltpu.VMEM((K, LANES), jnp.float32)] * 2,
            [pltpu.VMEM((K, LANES), jnp.float32)] * 2,
            [pltpu.SemaphoreType.DMA] * 2,
            [pltpu.SemaphoreType.DMA] * 2,
            [pltpu.SemaphoreType.DMA] * 2,
        ],
    )
    def k(src_h, dst_h, rel_h, ones_h, cnt0_h, cnt1_h, gidx_h, norm_h,
          srcb, dstb, relb, keyb, onesb, c0b, c1b, isem, gsem, osem):
        c = lax.axis_index("c")
        s = lax.axis_index("s")
        w = s * NC + c

        def fire_idx(i, b):
            r0 = w * RW + i * K
            pltpu.async_copy(src_h.at[pl.ds(r0, K)], srcb[b], isem[b])
            pltpu.async_copy(dst_h.at[pl.ds(r0, K)], dstb[b], isem[b])
            pltpu.async_copy(rel_h.at[pl.ds(r0, K)], relb[b], isem[b])
            pltpu.async_copy(ones_h.at[pl.ds(r0, K)], onesb[b], isem[b])

        def wait_idx(b):
            for _ in range(4):
                pltpu.make_async_copy(src_h.at[pl.ds(0, K)], srcb[b],
                                      isem[b]).wait()

        def compute_keys(b):
            for j in range(K):
                for t in range(LANES // 16):
                    sl = pl.ds(t * 16, 16)
                    rv = relb[b][j, sl]
                    keyb[b][j, sl] = dstb[b][j, sl] * NREL + rv
                    srcb[b][j, sl] = ((rv >> 2) * (4 * n_nodes)
                                      + srcb[b][j, sl] * 4 + (rv & 3))

        def fire_gathers(b):
            for j in range(K):
                pltpu.async_copy(cnt0_h.at[keyb[b].at[j]], c0b[b].at[j],
                                 gsem[b])
                pltpu.async_copy(cnt1_h.at[keyb[b].at[j]], c1b[b].at[j],
                                 gsem[b])

        def drain_gathers(b):
            for j in range(K):
                pltpu.make_async_copy(cnt0_h.at[keyb[b].at[j]],
                                      c0b[b].at[j], gsem[b]).wait()
                pltpu.make_async_copy(cnt1_h.at[keyb[b].at[j]],
                                      c1b[b].at[j], gsem[b]).wait()

        def compute_norm(b):
            for j in range(K):
                for t in range(LANES // 16):
                    sl = pl.ds(t * 16, 16)
                    tot = jnp.maximum(c0b[b][j, sl] + c1b[b][j, sl], 1.0)
                    onesb[b][j, sl] = onesb[b][j, sl] / tot

        def fire_out(i, b):
            r0 = w * RW + i * K
            pltpu.async_copy(srcb[b], gidx_h.at[pl.ds(r0, K)], osem[b])
            pltpu.async_copy(onesb[b], norm_h.at[pl.ds(r0, K)], osem[b])

        def drain_out(b):
            for _ in range(2):
                pltpu.make_async_copy(srcb[b], gidx_h.at[pl.ds(0, K)],
                                      osem[b]).wait()

        fire_idx(0, 0)
        wait_idx(0)
        compute_keys(0)
        fire_gathers(0)
        fire_idx(1, 1)
        drain_gathers(0)
        compute_norm(0)
        fire_out(0, 0)

        def steady(i, cur):
            nxt = 1 - cur
            wait_idx(cur)
            compute_keys(cur)
            fire_gathers(cur)
            drain_out(nxt)
            fire_idx(i + 1, nxt)
            drain_gathers(cur)
            compute_norm(cur)
            fire_out(i, cur)

        def pair(p, _):
            steady(1 + 2 * p, 1)
            steady(2 + 2 * p, 0)
            return _
        lax.fori_loop(0, (NI - 2) // 2, pair, None)

        wait_idx(1)
        compute_keys(1)
        fire_gathers(1)
        drain_out(0)
        drain_gathers(1)
        compute_norm(1)
        fire_out(NI - 1, 1)
        drain_out(1)

    return k(src2, dst2, rel2, ones2, cnt0, cnt1)


def _sc_scatter(y, gidx2, dst2, norm2, n_nodes):
    KS = 2
    RT = gidx2.shape[0]
    RW = RT // NW
    NI = RW // KS
    C = y.shape[1]
    rows_tile = -(-n_nodes // (NS * 8)) * 8
    n_pad = rows_tile * NS
    ZR = next(z for z in range(256, 7, -8) if rows_tile % z == 0)
    nz = rows_tile // ZR

    @functools.partial(
        pl.kernel,
        out_type=jax.ShapeDtypeStruct((NC * n_pad, C), jnp.float32),
        mesh=_mesh(),
        compiler_params=pltpu.CompilerParams(use_tc_tiling_on_sc=False),
        scratch_types=[
            [pltpu.VMEM((KS, LANES), jnp.int32)] * 2,
            [pltpu.VMEM((KS, LANES), jnp.int32)] * 2,
            [pltpu.VMEM((KS, LANES), jnp.float32)] * 2,
            [pltpu.VMEM((KS * LANES, C), jnp.float32)] * 2,
            pltpu.VMEM((ZR, C), jnp.float32),
            pltpu.VMEM_SHARED((n_pad, C), jnp.float32),
            [pltpu.SemaphoreType.DMA] * 2,
            [pltpu.SemaphoreType.DMA] * 2,
            [pltpu.SemaphoreType.DMA] * 2,
        ],
    )
    def k(y_h, gidx_h, dst_h, norm_h, out_h, gidxb, dstb, normb, msgb, zb,
          acc, isem, gsem, ssem):
        c = lax.axis_index("c")
        s = lax.axis_index("s")
        w = s * NC + c

        def zrow(j, _):
            zb[j, pl.ds(0, 16)] = jnp.zeros((16,), jnp.float32)
            zb[j, pl.ds(16, 16)] = jnp.zeros((16,), jnp.float32)
            return _
        lax.fori_loop(0, ZR, zrow, None)

        def zacc(q, _):
            pltpu.sync_copy(zb, acc.at[pl.ds(s * rows_tile + q * ZR, ZR)])
            return _
        lax.fori_loop(0, nz, zacc, None)
        plsc.subcore_barrier()

        def fire_idx(i, b):
            r0 = w * RW + i * KS
            pltpu.async_copy(gidx_h.at[pl.ds(r0, KS)], gidxb[b], isem[b])
            pltpu.async_copy(dst_h.at[pl.ds(r0, KS)], dstb[b], isem[b])
            pltpu.async_copy(norm_h.at[pl.ds(r0, KS)], normb[b], isem[b])

        def wait_idx(b):
            pltpu.make_async_copy(gidx_h.at[pl.ds(0, KS)], gidxb[b],
                                  isem[b]).wait()
            pltpu.make_async_copy(dst_h.at[pl.ds(0, KS)], dstb[b],
                                  isem[b]).wait()
            pltpu.make_async_copy(norm_h.at[pl.ds(0, KS)], normb[b],
                                  isem[b]).wait()

        def fire_gathers(b):
            for j in range(KS):
                pltpu.async_copy(y_h.at[gidxb[b].at[j]],
                                 msgb[b].at[pl.ds(j * LANES, LANES)], gsem[b])

        def drain_gathers(b):
            for j in range(KS):
                pltpu.make_async_copy(
                    y_h.at[gidxb[b].at[j]],
                    msgb[b].at[pl.ds(j * LANES, LANES)], gsem[b]).wait()

        def fire_scatters(b):
            for j in range(KS):
                pltpu.async_copy(msgb[b].at[pl.ds(j * LANES, LANES)],
                                 acc.at[dstb[b].at[j]], ssem[b], add=True)

        def drain_scatters(b):
            for j in range(KS):
                pltpu.make_async_copy(msgb[b].at[pl.ds(j * LANES, LANES)],
                                      acc.at[dstb[b].at[j]], ssem[b]).wait()

        def scale(b):
            for j in range(KS):
                @plsc.parallel_loop(0, LANES // 16, 1, unroll=2)
                def scale_t(t):
                    nv16 = normb[b][j, pl.ds(t * 16, 16)]
                    for l in range(16):
                        nv = nv16[l]
                        row = j * LANES + t * 16 + l
                        msgb[b][row, pl.ds(0, 16)] = (
                            msgb[b][row, pl.ds(0, 16)] * nv)
                        msgb[b][row, pl.ds(16, 16)] = (
                            msgb[b][row, pl.ds(16, 16)] * nv)

        fire_idx(0, 0)
        wait_idx(0)
        fire_gathers(0)

        fire_idx(1, 1)
        drain_gathers(0)
        wait_idx(1)
        fire_gathers(1)
        scale(0)
        fire_scatters(0)

        def steady(i, cur):
            nxt = 1 - cur
            drain_scatters(nxt)
            fire_idx(i + 1, nxt)
            drain_gathers(cur)
            wait_idx(nxt)
            fire_gathers(nxt)
            scale(cur)
            fire_scatters(cur)

        def pair(p, _):
            steady(1 + 2 * p, 1)
            steady(2 + 2 * p, 0)
            return _
        lax.fori_loop(0, (NI - 2) // 2, pair, None)

        drain_scatters(0)
        drain_gathers(1)
        scale(1)
        fire_scatters(1)
        drain_scatters(1)
        plsc.subcore_barrier()

        def flush(q, _):
            row = s * rows_tile + q * ZR
            pltpu.sync_copy(acc.at[pl.ds(row, ZR)], zb)
            pltpu.sync_copy(zb, out_h.at[pl.ds(c * n_pad + row, ZR)])
            return _
        lax.fori_loop(0, nz, flush, None)

    out = k(y, gidx2, dst2, norm2)
    return out[:n_nodes], out[n_pad:n_pad + n_nodes]


def kernel(embedding, edge_index, edge_type, in_proj_w, in_proj_b,
           out_proj_w, out_proj_b, w1, root1, b1, w2, root2, b2):
    N = embedding.shape[1]
    D = embedding.shape[2]
    E = edge_index.shape[1]
    x = embedding[0]

    w0 = in_proj_w[2 * D:3 * D].T @ out_proj_w.T
    b0 = (in_proj_b[2 * D:3 * D] @ out_proj_w.T + out_proj_b).reshape(1, D)
    r1eff = w0 @ root1
    c1 = b0 @ root1 + b1.reshape(1, D)
    w1eff = jnp.einsum('ab,rbc->rac', w0, w1)
    c1r = jnp.einsum('ab,rbc->rac', b0, w1)

    block = NW * LANES * K
    RT = -(-E // block) * (block // LANES)
    EP = RT * LANES
    pad = EP - E
    src = jnp.pad(edge_index[0], (0, pad)).reshape(RT, LANES)
    dst = jnp.pad(edge_index[1], (0, pad)).reshape(RT, LANES)
    rel = jnp.pad(edge_type, (0, pad)).reshape(RT, LANES)
    ones = jnp.pad(jnp.ones((E,), jnp.float32), (0, pad)).reshape(RT, LANES)

    nbins = N * NREL
    cnt = _sc_hist(dst, rel, ones, nbins)
    gidx, norm = _sc_norm(src, dst, rel, ones, cnt[:nbins], cnt[nbins:], N)

    base1, y10, y11, y12 = _dense1(x, r1eff, c1, w1eff, c1r)

    p0a, p0b = _sc_scatter(y10.reshape(N * NREL, 32), gidx, dst, norm, N)
    p1a, p1b = _sc_scatter(y11.reshape(N * NREL, 32), gidx, dst, norm, N)
    p2a, p2b = _sc_scatter(y12.reshape(N * NREL, 32), gidx, dst, norm, N)

    y2, base2 = _dense2(base1, p0a, p0b, p1a, p1b, p2a, p2b,
                        w2, root2, b2.reshape(1, 32))

    qa, qb = _sc_scatter(y2.reshape(N * NREL, 32), gidx, dst, norm, N)
    return _final(base2, qa, qb)


# speedup vs baseline: 29.9110x; 1.0108x over previous
import functools

import jax
import jax.numpy as jnp
from jax import lax
from jax.experimental import pallas as pl
from jax.experimental.pallas import tpu as pltpu
from jax.experimental.pallas import tpu_sc as plsc

NREL = 8
NC = 2
NS = 16
NW = NC * NS
LANES = 128
K = 7


def _mesh():
    return plsc.VectorSubcoreMesh(
        core_axis_name="c", subcore_axis_name="s", num_cores=NC,
        num_subcores=NS)


def _dense1(x, r1eff, c1, w1eff, c1r):
    N, D = x.shape
    BN = 2000
    G = N // BN

    def body(x_ref, r1_ref, c1_ref, w1_ref, c1r_ref, base_ref,
             y0_ref, y1_ref, y2_ref):
        xb = x_ref[...]
        base_ref[...] = (
            jnp.dot(xb, r1_ref[...], preferred_element_type=jnp.float32)
            + c1_ref[...])
        z = [jnp.dot(xb, w1_ref[r], preferred_element_type=jnp.float32)
             + c1r_ref[r] for r in range(NREL)]
        for p in range(2):
            y0_ref[p] = jnp.concatenate(
                [z[4 * p + q][:, 0:32] for q in range(4)], axis=1)
            y1_ref[p] = jnp.concatenate(
                [z[4 * p + q][:, 32:64] for q in range(4)], axis=1)
            y2_ref[p] = jnp.concatenate(
                [z[4 * p + q][:, 64:96] for q in range(4)], axis=1)

    full2 = lambda shape: pl.BlockSpec(shape, lambda i: (0, 0))
    yspec = pl.BlockSpec((2, BN, 128), lambda i: (0, i, 0))
    yshape = jax.ShapeDtypeStruct((2, N, 128), jnp.float32)
    out = pl.pallas_call(
        body,
        grid=(G,),
        in_specs=[
            pl.BlockSpec((BN, D), lambda i: (i, 0)),
            full2((D, D)), full2((1, D)),
            pl.BlockSpec((NREL, D, D), lambda i: (0, 0, 0)),
            pl.BlockSpec((NREL, 1, D), lambda i: (0, 0, 0)),
        ],
        out_specs=[
            pl.BlockSpec((BN, D), lambda i: (i, 0)),
            yspec, yspec, yspec,
        ],
        out_shape=[
            jax.ShapeDtypeStruct((N, D), jnp.float32),
            yshape, yshape, yshape,
        ],
    )(x, r1eff, c1, w1eff, c1r)
    return out


def _dense2(base1, p00, p10, p01, p11, p02, p12, w2, root2, b2):
    N, D = base1.shape
    C = root2.shape[1]
    BN = 2000
    G = N // BN

    def body(base_ref, a0_ref, b0_ref, a1_ref, b1_ref, a2_ref, b2p_ref,
             w2_ref, r2_ref, b2_ref, y_ref, base2_ref):
        b = base_ref[...]
        h0 = jnp.maximum(b[:, 0:32] + a0_ref[...] + b0_ref[...], 0.0)
        h1 = jnp.maximum(b[:, 32:64] + a1_ref[...] + b1_ref[...], 0.0)
        h2 = jnp.maximum(b[:, 64:96] + a2_ref[...] + b2p_ref[...], 0.0)
        r2 = r2_ref[...]
        base2_ref[...] = (
            jnp.dot(h0, r2[0:32], preferred_element_type=jnp.float32)
            + jnp.dot(h1, r2[32:64], preferred_element_type=jnp.float32)
            + jnp.dot(h2, r2[64:96], preferred_element_type=jnp.float32)
            + b2_ref[...])
        z = []
        for r in range(NREL):
            w = w2_ref[r]
            z.append(
                jnp.dot(h0, w[0:32], preferred_element_type=jnp.float32)
                + jnp.dot(h1, w[32:64], preferred_element_type=jnp.float32)
                + jnp.dot(h2, w[64:96], preferred_element_type=jnp.float32))
        for p in range(2):
            y_ref[p] = jnp.concatenate(
                [z[4 * p + q] for q in range(4)], axis=1)

    chunk = pl.BlockSpec((BN, 32), lambda i: (i, 0))
    out = pl.pallas_call(
        body,
        grid=(G,),
        in_specs=[
            pl.BlockSpec((BN, D), lambda i: (i, 0)),
            chunk, chunk, chunk, chunk, chunk, chunk,
            pl.BlockSpec((NREL, D, C), lambda i: (0, 0, 0)),
            pl.BlockSpec((D, C), lambda i: (0, 0)),
            pl.BlockSpec((1, C), lambda i: (0, 0)),
        ],
        out_specs=[
            pl.BlockSpec((2, BN, 128), lambda i: (0, i, 0)),
            pl.BlockSpec((BN, C), lambda i: (i, 0)),
        ],
        out_shape=[
            jax.ShapeDtypeStruct((2, N, 128), jnp.float32),
            jax.ShapeDtypeStruct((N, C), jnp.float32),
        ],
    )(base1, p00, p10, p01, p11, p02, p12, w2, root2, b2)
    return out


def _final(base2, q0, q1):
    N, C = base2.shape
    BN = 2000
    G = N // BN

    def body(b_ref, q0_ref, q1_ref, o_ref):
        o_ref[...] = jax.nn.sigmoid(b_ref[...] + q0_ref[...] + q1_ref[...])

    spec = pl.BlockSpec((BN, C), lambda i: (i, 0))
    return pl.pallas_call(
        body,
        grid=(G,),
        in_specs=[spec, spec, spec],
        out_specs=spec,
        out_shape=jax.ShapeDtypeStruct((N, C), jnp.float32),
    )(base2, q0, q1)


def _sc_hist(dst2, rel2, ones2, nbins):
    RT = dst2.shape[0]
    RW = RT // NW
    NI = RW // K
    per_tile = nbins // NS
    SB = 2000

    @functools.partial(
        pl.kernel,
        out_type=jax.ShapeDtypeStruct((NC * nbins,), jnp.float32),
        mesh=_mesh(),
        compiler_params=pltpu.CompilerParams(use_tc_tiling_on_sc=False),
        scratch_types=[
            [pltpu.VMEM((K, LANES), jnp.int32)] * 2,
            [pltpu.VMEM((K, LANES), jnp.int32)] * 2,
            [pltpu.VMEM((K, LANES), jnp.int32)] * 2,
            [pltpu.VMEM((K, LANES), jnp.float32)] * 2,
            pltpu.VMEM((SB,), jnp.float32),
            pltpu.VMEM_SHARED((nbins,), jnp.float32),
            [pltpu.SemaphoreType.DMA] * 2,
            [pltpu.SemaphoreType.DMA] * 2,
        ],
    )
    def k(dst_h, rel_h, ones_h, out_h, dstb, relb, keyb, onesb, stage, cnt,
          isem, ssem):
        c = lax.axis_index("c")
        s = lax.axis_index("s")
        w = s * NC + c

        def zstage(i, _):
            stage[pl.ds(i * 16, 16)] = jnp.zeros((16,), jnp.float32)
            return _
        lax.fori_loop(0, SB // 16, zstage, None)

        nfull = per_tile // SB
        rem = per_tile - nfull * SB

        def zbin(q, _):
            pltpu.sync_copy(stage, cnt.at[pl.ds(s * per_tile + q * SB, SB)])
            return _
        lax.fori_loop(0, nfull, zbin, None)
        if rem:
            pltpu.sync_copy(stage.at[pl.ds(0, rem)],
                            cnt.at[pl.ds(s * per_tile + nfull * SB, rem)])
        plsc.subcore_barrier()

        def fire_idx(i, b):
            r0 = w * RW + i * K
            pltpu.async_copy(dst_h.at[pl.ds(r0, K)], dstb[b], isem[b])
            pltpu.async_copy(rel_h.at[pl.ds(r0, K)], relb[b], isem[b])
            pltpu.async_copy(ones_h.at[pl.ds(r0, K)], onesb[b], isem[b])

        def wait_idx(b):
            for _ in range(3):
                pltpu.make_async_copy(dst_h.at[pl.ds(0, K)], dstb[b],
                                      isem[b]).wait()

        def compute_keys(b):
            for j in range(K):
                for t in range(LANES // 16):
                    sl = pl.ds(t * 16, 16)
                    keyb[b][j, sl] = dstb[b][j, sl] * NREL + relb[b][j, sl]

        def fire_adds(b):
            for j in range(K):
                pltpu.async_copy(onesb[b].at[j], cnt.at[keyb[b].at[j]],
                                 ssem[b], add=True)

        def drain_adds(b):
            for j in range(K):
                pltpu.make_async_copy(onesb[b].at[j], cnt.at[keyb[b].at[j]],
                                      ssem[b]).wait()

        fire_idx(0, 0)
        fire_idx(1, 1)
        wait_idx(0)
        compute_keys(0)
        fire_adds(0)

        def steady(i, cur):
            nxt = 1 - cur
            wait_idx(cur)
            compute_keys(cur)
            drain_adds(nxt)
            fire_idx(i + 1, nxt)
            fire_adds(cur)

        def pair(p, _):
            steady(1 + 2 * p, 1)
            steady(2 + 2 * p, 0)
            return _
        lax.fori_loop(0, (NI - 2) // 2, pair, None)

        wait_idx(1)
        compute_keys(1)
        drain_adds(0)
        fire_adds(1)
        drain_adds(1)
        plsc.subcore_barrier()

        def flush(q, _):
            off = s * per_tile + q * SB
            pltpu.sync_copy(cnt.at[pl.ds(off, SB)], stage)
            pltpu.sync_copy(stage, out_h.at[pl.ds(c * nbins + off, SB)])
            return _
        lax.fori_loop(0, nfull, flush, None)
        if rem:
            off = s * per_tile + nfull * SB
            pltpu.sync_copy(cnt.at[pl.ds(off, rem)], stage.at[pl.ds(0, rem)])
            pltpu.sync_copy(stage.at[pl.ds(0, rem)],
                            out_h.at[pl.ds(c * nbins + off, rem)])

    return k(dst2, rel2, ones2)


def _sc_norm(src2, dst2, rel2, ones2, cnt0, cnt1, n_nodes):
    RT = src2.shape[0]
    RW = RT // NW
    NI = RW // K

    @functools.partial(
        pl.kernel,
        out_type=(
            jax.ShapeDtypeStruct((RT, LANES), jnp.int32),
            jax.ShapeDtypeStruct((RT, LANES), jnp.float32),
        ),
        mesh=_mesh(),
        compiler_params=pltpu.CompilerParams(use_tc_tiling_on_sc=False),
        scratch_types=[
            [pltpu.VMEM((K, LANES), jnp.int32)] * 2,
            [pltpu.VMEM((K, LANES), jnp.int32)] * 2,
            [pltpu.VMEM((K, LANES), jnp.int32)] * 2,
            [pltpu.VMEM((K, LANES), jnp.int32)] * 2,
            [pltpu.VMEM((K, LANES), jnp.float32)] * 2,
            [pltpu.VMEM((K, LANES), jnp.float32)] * 2,
            [pltpu.VMEM((K, LANES), jnp.float32)] * 2,
            [pltpu.SemaphoreType.DMA] * 2,
            [pltpu.SemaphoreType.DMA] * 2,
            [pltpu.SemaphoreType.DMA] * 2,
        ],
    )
    def k(src_h, dst_h, rel_h, ones_h, cnt0_h, cnt1_h, gidx_h, norm_h,
          srcb, dstb, relb, keyb, onesb, c0b, c1b, isem, gsem, osem):
        c = lax.axis_index("c")
        s = lax.axis_index("s")
        w = s * NC + c

        def fire_idx(i, b):
            r0 = w * RW + i * K
            pltpu.async_copy(src_h.at[pl.ds(r0, K)], srcb[b], isem[b])
            pltpu.async_copy(dst_h.at[pl.ds(r0, K)], dstb[b], isem[b])
            pltpu.async_copy(rel_h.at[pl.ds(r0, K)], relb[b], isem[b])
            pltpu.async_copy(ones_h.at[pl.ds(r0, K)], onesb[b], isem[b])

        def wait_idx(b):
            for _ in range(4):
                pltpu.make_async_copy(src_h.at[pl.ds(0, K)], srcb[b],
                                      isem[b]).wait()

        def compute_keys(b):
            for j in range(K):
                for t in range(LANES // 16):
                    sl = pl.ds(t * 16, 16)
                    rv = relb[b][j, sl]
                    keyb[b][j, sl] = dstb[b][j, sl] * NREL + rv
                    srcb[b][j, sl] = ((rv >> 2) * (4 * n_nodes)
                                      + srcb[b][j, sl] * 4 + (rv & 3))

        def fire_gathers(b):
            for j in range(K):
                pltpu.async_copy(cnt0_h.at[keyb[b].at[j]], c0b[b].at[j],
                                 gsem[b])
                pltpu.async_copy(cnt1_h.at[keyb[b].at[j]], c1b[b].at[j],
                                 gsem[b])

        def drain_gathers(b):
            for j in range(K):
                pltpu.make_async_copy(cnt0_h.at[keyb[b].at[j]],
                                      c0b[b].at[j], gsem[b]).wait()
                pltpu.make_async_copy(cnt1_h.at[keyb[b].at[j]],
                                      c1b[b].at[j], gsem[b]).wait()

        def compute_norm(b):
            for j in range(K):
                for t in range(LANES // 16):
                    sl = pl.ds(t * 16, 16)
                    tot = jnp.maximum(c0b[b][j, sl] + c1b[b][j, sl], 1.0)
                    onesb[b][j, sl] = onesb[b][j, sl] / tot

        def fire_out(i, b):
            r0 = w * RW + i * K
            pltpu.async_copy(srcb[b], gidx_h.at[pl.ds(r0, K)], osem[b])
            pltpu.async_copy(onesb[b], norm_h.at[pl.ds(r0, K)], osem[b])

        def drain_out(b):
            for _ in range(2):
                pltpu.make_async_copy(srcb[b], gidx_h.at[pl.ds(0, K)],
                                      osem[b]).wait()

        fire_idx(0, 0)
        wait_idx(0)
        compute_keys(0)
        fire_gathers(0)
        fire_idx(1, 1)
        drain_gathers(0)
        compute_norm(0)
        fire_out(0, 0)

        def steady(i, cur):
            nxt = 1 - cur
            wait_idx(cur)
            compute_keys(cur)
            fire_gathers(cur)
            drain_out(nxt)
            fire_idx(i + 1, nxt)
            drain_gathers(cur)
            compute_norm(cur)
            fire_out(i, cur)

        def pair(p, _):
            steady(1 + 2 * p, 1)
            steady(2 + 2 * p, 0)
            return _
        lax.fori_loop(0, (NI - 2) // 2, pair, None)

        wait_idx(1)
        compute_keys(1)
        fire_gathers(1)
        drain_out(0)
        drain_gathers(1)
        compute_norm(1)
        fire_out(NI - 1, 1)
        drain_out(1)

    return k(src2, dst2, rel2, ones2, cnt0, cnt1)


def _sc_scatter(y, gidx2, dst2, norm2, n_nodes):
    KS = 2
    RT = gidx2.shape[0]
    RW = RT // NW
    NI = RW // KS
    C = y.shape[1]
    rows_tile = -(-n_nodes // (NS * 8)) * 8
    n_pad = rows_tile * NS
    ZR = next(z for z in range(256, 7, -8) if rows_tile % z == 0)
    nz = rows_tile // ZR

    @functools.partial(
        pl.kernel,
        out_type=jax.ShapeDtypeStruct((NC * n_pad, C), jnp.float32),
        mesh=_mesh(),
        compiler_params=pltpu.CompilerParams(use_tc_tiling_on_sc=False),
        scratch_types=[
            [pltpu.VMEM((KS, LANES), jnp.int32)] * 2,
            [pltpu.VMEM((KS, LANES), jnp.int32)] * 2,
            [pltpu.VMEM((KS, LANES), jnp.float32)] * 2,
            [pltpu.VMEM((KS * LANES, C), jnp.float32)] * 2,
            pltpu.VMEM((ZR, C), jnp.float32),
            pltpu.VMEM_SHARED((n_pad, C), jnp.float32),
            [pltpu.SemaphoreType.DMA] * 2,
            [pltpu.SemaphoreType.DMA] * 2,
            [pltpu.SemaphoreType.DMA] * 2,
        ],
    )
    def k(y_h, gidx_h, dst_h, norm_h, out_h, gidxb, dstb, normb, msgb, zb,
          acc, isem, gsem, ssem):
        c = lax.axis_index("c")
        s = lax.axis_index("s")
        w = s * NC + c

        def zrow(j, _):
            zb[j, pl.ds(0, 16)] = jnp.zeros((16,), jnp.float32)
            zb[j, pl.ds(16, 16)] = jnp.zeros((16,), jnp.float32)
            return _
        lax.fori_loop(0, ZR, zrow, None)

        for q in range(nz):
            pltpu.async_copy(zb, acc.at[pl.ds(s * rows_tile + q * ZR, ZR)],
                             gsem[0])
        for q in range(nz):
            pltpu.make_async_copy(
                zb, acc.at[pl.ds(s * rows_tile + q * ZR, ZR)], gsem[0]).wait()
        plsc.subcore_barrier()

        def fire_idx(i, b):
            r0 = w * RW + i * KS
            pltpu.async_copy(gidx_h.at[pl.ds(r0, KS)], gidxb[b], isem[b])
            pltpu.async_copy(dst_h.at[pl.ds(r0, KS)], dstb[b], isem[b])
            pltpu.async_copy(norm_h.at[pl.ds(r0, KS)], normb[b], isem[b])

        def wait_idx(b):
            pltpu.make_async_copy(gidx_h.at[pl.ds(0, KS)], gidxb[b],
                                  isem[b]).wait()
            pltpu.make_async_copy(dst_h.at[pl.ds(0, KS)], dstb[b],
                                  isem[b]).wait()
            pltpu.make_async_copy(norm_h.at[pl.ds(0, KS)], normb[b],
                                  isem[b]).wait()

        def fire_gathers(b):
            for j in range(KS):
                pltpu.async_copy(y_h.at[gidxb[b].at[j]],
                                 msgb[b].at[pl.ds(j * LANES, LANES)], gsem[b])

        def drain_gathers(b):
            for j in range(KS):
                pltpu.make_async_copy(
                    y_h.at[gidxb[b].at[j]],
                    msgb[b].at[pl.ds(j * LANES, LANES)], gsem[b]).wait()

        def fire_scatters(b):
            for j in range(KS):
                pltpu.async_copy(msgb[b].at[pl.ds(j * LANES, LANES)],
                                 acc.at[dstb[b].at[j]], ssem[b], add=True)

        def drain_scatters(b):
            for j in range(KS):
                pltpu.make_async_copy(msgb[b].at[pl.ds(j * LANES, LANES)],
                                      acc.at[dstb[b].at[j]], ssem[b]).wait()

        def scale(b):
            for j in range(KS):
                @plsc.parallel_loop(0, LANES // 16, 1, unroll=2)
                def scale_t(t):
                    nv16 = normb[b][j, pl.ds(t * 16, 16)]
                    for l in range(16):
                        nv = nv16[l]
                        row = j * LANES + t * 16 + l
                        msgb[b][row, pl.ds(0, 16)] = (
                            msgb[b][row, pl.ds(0, 16)] * nv)
                        msgb[b][row, pl.ds(16, 16)] = (
                            msgb[b][row, pl.ds(16, 16)] * nv)

        fire_idx(0, 0)
        wait_idx(0)
        fire_gathers(0)

        fire_idx(1, 1)
        drain_gathers(0)
        wait_idx(1)
        fire_gathers(1)
        scale(0)
        fire_scatters(0)

        def steady(i, cur):
            nxt = 1 - cur
            drain_scatters(nxt)
            fire_idx(i + 1, nxt)
            drain_gathers(cur)
            wait_idx(nxt)
            fire_gathers(nxt)
            scale(cur)
            fire_scatters(cur)

        def pair(p, _):
            steady(1 + 2 * p, 1)
            steady(2 + 2 * p, 0)
            return _
        lax.fori_loop(0, (NI - 2) // 2, pair, None)

        drain_scatters(0)
        drain_gathers(1)
        scale(1)
        fire_scatters(1)
        drain_scatters(1)
        plsc.subcore_barrier()

        stg = (zb, msgb[0].at[pl.ds(0, ZR)])

        def in_row(q):
            return acc.at[pl.ds(s * rows_tile + q * ZR, ZR)]

        def out_row(q):
            return out_h.at[pl.ds(c * n_pad + s * rows_tile + q * ZR, ZR)]

        pltpu.async_copy(in_row(0), stg[0], gsem[0])
        for q in range(nz):
            cur = stg[q % 2]
            pltpu.make_async_copy(in_row(q), cur, gsem[q % 2]).wait()
            if q >= 1:
                pltpu.make_async_copy(stg[(q - 1) % 2], out_row(q - 1),
                                      ssem[(q - 1) % 2]).wait()
            if q + 1 < nz:
                pltpu.async_copy(in_row(q + 1), stg[(q + 1) % 2],
                                 gsem[(q + 1) % 2])
            pltpu.async_copy(cur, out_row(q), ssem[q % 2])
        pltpu.make_async_copy(stg[(nz - 1) % 2], out_row(nz - 1),
                              ssem[(nz - 1) % 2]).wait()

    out = k(y, gidx2, dst2, norm2)
    return out[:n_nodes], out[n_pad:n_pad + n_nodes]


def kernel(embedding, edge_index, edge_type, in_proj_w, in_proj_b,
           out_proj_w, out_proj_b, w1, root1, b1, w2, root2, b2):
    N = embedding.shape[1]
    D = embedding.shape[2]
    E = edge_index.shape[1]
    x = embedding[0]

    w0 = in_proj_w[2 * D:3 * D].T @ out_proj_w.T
    b0 = (in_proj_b[2 * D:3 * D] @ out_proj_w.T + out_proj_b).reshape(1, D)
    r1eff = w0 @ root1
    c1 = b0 @ root1 + b1.reshape(1, D)
    w1eff = jnp.einsum('ab,rbc->rac', w0, w1)
    c1r = jnp.einsum('ab,rbc->rac', b0, w1)

    block = NW * LANES * K
    RT = -(-E // block) * (block // LANES)
    EP = RT * LANES
    pad = EP - E
    src = jnp.pad(edge_index[0], (0, pad)).reshape(RT, LANES)
    dst = jnp.pad(edge_index[1], (0, pad)).reshape(RT, LANES)
    rel = jnp.pad(edge_type, (0, pad)).reshape(RT, LANES)
    ones = jnp.pad(jnp.ones((E,), jnp.float32), (0, pad)).reshape(RT, LANES)

    nbins = N * NREL
    cnt = _sc_hist(dst, rel, ones, nbins)
    gidx, norm = _sc_norm(src, dst, rel, ones, cnt[:nbins], cnt[nbins:], N)

    base1, y10, y11, y12 = _dense1(x, r1eff, c1, w1eff, c1r)

    p0a, p0b = _sc_scatter(y10.reshape(N * NREL, 32), gidx, dst, norm, N)
    p1a, p1b = _sc_scatter(y11.reshape(N * NREL, 32), gidx, dst, norm, N)
    p2a, p2b = _sc_scatter(y12.reshape(N * NREL, 32), gidx, dst, norm, N)

    y2, base2 = _dense2(base1, p0a, p0b, p1a, p1b, p2a, p2b,
                        w2, root2, b2.reshape(1, 32))

    qa, qb = _sc_scatter(y2.reshape(N * NREL, 32), gidx, dst, norm, N)
    return _final(base2, qa, qb)
